# Initial kernel scaffold; baseline (speedup 1.0000x reference)
import jax
import jax.numpy as jnp
from jax.experimental import pallas as pl


def kernel(x, edge_index, batch, W1, asrc1, adst1, b1, W2, asrc2, adst2, b2, W3, asrc3, adst3, b3, p1, p2, lw1, lb1, lw2, lb2, lw3, lb3):
    raise NotImplementedError("write your pallas kernel here")


# speedup vs baseline: 29.9368x; 29.9368x over previous
import functools
import math

import jax
import jax.numpy as jnp
from jax import lax
from jax.experimental import pallas as pl
from jax.experimental.pallas import tpu as pltpu
from jax.experimental.pallas import tpu_sc as plsc

N = 10000
D = 128
E = 320000
NUM_CLASSES = 10

NPAD = 10240
NC, NS, L = 2, 16, 16
NW = NC * NS
C = 128
EPAD = ((E + NW * C - 1) // (NW * C)) * (NW * C)
NCHUNK = EPAD // (NW * C)
ETILE = NCHUNK * C
ROWS_T = NPAD // NS
NEG = -1e9

f32 = jnp.float32


def _pre_body(x_ref, w_ref, avs_ref, avd_ref, act_ref,
              h_ref, asm_ref, adm_ref, amax_ref):
    x = x_ref[...]
    h = jnp.dot(x, w_ref[...], preferred_element_type=f32)
    h_ref[...] = h
    a_s = jnp.dot(h, avs_ref[...], preferred_element_type=f32)
    a_d = jnp.dot(h, avd_ref[...], preferred_element_type=f32)
    act = act_ref[...]
    asm = jnp.where(act > 0, a_s, NEG)
    adm = jnp.where(act > 0, a_d, NEG)
    asm_ref[...] = asm
    adm_ref[...] = adm
    amax_ref[...] = jnp.full((1, 128), jnp.max(asm), dtype=f32)


def _pre_call(X, W, avs, avd, act):
    return pl.pallas_call(
        _pre_body,
        out_shape=[
            jax.ShapeDtypeStruct((NPAD, D), f32),
            jax.ShapeDtypeStruct((NPAD, 1), f32),
            jax.ShapeDtypeStruct((NPAD, 1), f32),
            jax.ShapeDtypeStruct((1, 128), f32),
        ],
    )(X, W, avs, avd, act)


def _edge_body(h_hbm, asm_hbm, adm_hbm, src_hbm, dst_hbm, amax_hbm,
               acc_out, den_out,
               as_v, ad_v, src_v, dst_v, rows_v, ex_v, zv, amax_v,
               acc_sh, den_sh, sem):
    cid = lax.axis_index("c")
    sid = lax.axis_index("s")
    wid = sid * NC + cid

    pltpu.sync_copy(asm_hbm, as_v)
    pltpu.sync_copy(adm_hbm, ad_v)
    pltpu.sync_copy(amax_hbm, amax_v)

    z16 = jnp.zeros((16,), f32)

    def _zrow(r, carry):
        for v in range(8):
            rows_v[r, pl.ds(16 * v, 16)] = z16
        return carry
    lax.fori_loop(0, C, _zrow, 0)

    def _zz(i, carry):
        zv[pl.ds(i * 16, 16)] = z16
        return carry
    lax.fori_loop(0, ROWS_T // 16, _zz, 0)

    base = sid * ROWS_T

    def _zacc(j, carry):
        pltpu.sync_copy(rows_v, acc_sh.at[pl.ds(base + j * C, C)])
        return carry
    lax.fori_loop(0, ROWS_T // C, _zacc, 0)
    pltpu.sync_copy(zv, den_sh.at[pl.ds(base, ROWS_T)])
    plsc.subcore_barrier()

    amax = amax_v[...]

    def _chunk(j, carry):
        pltpu.sync_copy(src_hbm.at[wid, j], src_v)
        pltpu.sync_copy(dst_hbm.at[wid, j], dst_v)
        pltpu.async_copy(h_hbm.at[src_v], rows_v, sem).wait()

        for i in range(C // 16):
            sv = src_v[pl.ds(i * 16, 16)]
            dv = dst_v[pl.ds(i * 16, 16)]
            a_s = plsc.load_gather(as_v, [sv])
            a_d = plsc.load_gather(ad_v, [dv])
            z = a_s + a_d
            lg = jnp.where(z >= 0, z, 0.2 * z)
            zb = amax + a_d
            m = jnp.where(zb >= 0, zb, 0.2 * zb)
            ex = jnp.exp(lg - m)
            ex_v[pl.ds(i * 16, 16)] = ex

        pltpu.sync_copy(ex_v, den_sh.at[dst_v], add=True)

        def _rower(r, rcarry):
            exr = plsc.load_gather(ex_v, [lax.broadcast(r, (16,))])
            for v in range(8):
                sl = pl.ds(16 * v, 16)
                rows_v[r, sl] = rows_v[r, sl] * exr
            return rcarry
        lax.fori_loop(0, C, _rower, 0)

        pltpu.sync_copy(rows_v, acc_sh.at[dst_v], add=True)
        return carry

    lax.fori_loop(0, NCHUNK, _chunk, 0)
    plsc.subcore_barrier()

    pltpu.sync_copy(acc_sh.at[pl.ds(base, ROWS_T)],
                    acc_out.at[cid, pl.ds(base, ROWS_T)])
    pltpu.sync_copy(den_sh.at[pl.ds(base, ROWS_T)],
                    den_out.at[cid, pl.ds(base, ROWS_T)])


def _edge_call(h, asm_flat, adm_flat, src3, dst3, amax16):
    mesh = plsc.VectorSubcoreMesh(core_axis_name="c", subcore_axis_name="s",
                                  num_cores=NC, num_subcores=NS)
    fn = pl.kernel(
        _edge_body,
        out_type=[
            jax.ShapeDtypeStruct((NC, NPAD, D), f32),
            jax.ShapeDtypeStruct((NC, NPAD), f32),
        ],
        mesh=mesh,
        scratch_types=[
            pltpu.VMEM((NPAD,), f32),
            pltpu.VMEM((NPAD,), f32),
            pltpu.VMEM((C,), jnp.int32),
            pltpu.VMEM((C,), jnp.int32),
            pltpu.VMEM((C, D), f32),
            pltpu.VMEM((C,), f32),
            pltpu.VMEM((ROWS_T,), f32),
            pltpu.VMEM((16,), f32),
            pltpu.VMEM_SHARED((NPAD, D), f32),
            pltpu.VMEM_SHARED((NPAD,), f32),
            pltpu.SemaphoreType.DMA,
        ],
        compiler_params=pltpu.CompilerParams(needs_layout_passes=False,
                                             use_tc_tiling_on_sc=False),
    )
    return fn(h, asm_flat, adm_flat, src3, dst3, amax16)


def _comb_body(acc_ref, den_ref, h_ref, asm_ref, adm_ref, act_ref,
               amax_ref, b_ref, p_ref, xp_ref, sc_ref, ms_ref):
    amax = amax_ref[0, 0]
    asm = asm_ref[...]
    adm = adm_ref[...]
    z = asm + adm
    lg = jnp.where(z >= 0, z, 0.2 * z)
    zb = amax + adm
    m = jnp.where(zb >= 0, zb, 0.2 * zb)
    exl = jnp.exp(lg - m)
    acc = acc_ref[0] + acc_ref[1]
    den = den_ref[0] + den_ref[1] + exl + 1e-16
    h = h_ref[...]
    out = (acc + exl * h) / den + b_ref[...]
    xp = jnp.maximum(out, 0.0)
    xp_ref[...] = xp
    p = p_ref[...]
    pn = jnp.sqrt(jnp.sum(p * p)) + 1e-16
    s = jnp.dot(xp, p, preferred_element_type=f32) / pn
    sc_ref[...] = s
    ms_ref[...] = jnp.where(act_ref[...] > 0, s, NEG)


def _comb_call(acc2, den2c, h, asm, adm, act, amax, b2d, pcol):
    BR = 2048
    G = NPAD // BR
    return pl.pallas_call(
        _comb_body,
        grid=(G,),
        in_specs=[
            pl.BlockSpec((NC, BR, D), lambda i: (0, i, 0)),
            pl.BlockSpec((NC, BR, 1), lambda i: (0, i, 0)),
            pl.BlockSpec((BR, D), lambda i: (i, 0)),
            pl.BlockSpec((BR, 1), lambda i: (i, 0)),
            pl.BlockSpec((BR, 1), lambda i: (i, 0)),
            pl.BlockSpec((BR, 1), lambda i: (i, 0)),
            pl.BlockSpec((1, 128), lambda i: (0, 0)),
            pl.BlockSpec((1, D), lambda i: (0, 0)),
            pl.BlockSpec((D, 1), lambda i: (0, 0)),
        ],
        out_specs=[
            pl.BlockSpec((BR, D), lambda i: (i, 0)),
            pl.BlockSpec((BR, 1), lambda i: (i, 0)),
            pl.BlockSpec((BR, 1), lambda i: (i, 0)),
        ],
        out_shape=[
            jax.ShapeDtypeStruct((NPAD, D), f32),
            jax.ShapeDtypeStruct((NPAD, 1), f32),
            jax.ShapeDtypeStruct((NPAD, 1), f32),
        ],
    )(acc2, den2c, h, asm, adm, act, amax, b2d, pcol)


def _thr_body(ms_ref, thr_ref, *, k):
    ms = ms_ref[...]
    lo0 = jnp.min(ms)
    hi0 = jnp.max(ms) + 1.0

    def _it(_, carry):
        lo, hi = carry
        mid = 0.5 * (lo + hi)
        cnt = jnp.sum(jnp.where(ms >= mid, 1.0, 0.0))
        ge = cnt >= k
        return (jnp.where(ge, mid, lo), jnp.where(ge, hi, mid))

    lo, hi = lax.fori_loop(0, 96, _it, (lo0, hi0))
    thr_ref[...] = jnp.full((1, 128), lo, dtype=f32)


def _thr_call(ms2d, k):
    return pl.pallas_call(
        functools.partial(_thr_body, k=k),
        out_shape=jax.ShapeDtypeStruct((1, 128), f32),
    )(ms2d)


def _pool_body(xp_ref, sc_ref, ms_ref, thr_ref,
               xn_ref, actn_ref, gmax_ref, gmean_ref, *, k, G):
    i = pl.program_id(0)
    thr = thr_ref[0, 0]
    kept = ms_ref[...] >= thr
    gate = jnp.where(kept, jnp.tanh(sc_ref[...]), 0.0)
    xn = xp_ref[...] * gate
    xn_ref[...] = xn
    actn_ref[...] = jnp.where(kept, 1.0, 0.0)
    xm = jnp.where(kept, xn, -1e30)
    bmax = jnp.max(xm, axis=0, keepdims=True)
    bsum = jnp.sum(xn, axis=0, keepdims=True)

    @pl.when(i == 0)
    def _():
        gmax_ref[...] = jnp.full((1, 128), -1e30, dtype=f32)
        gmean_ref[...] = jnp.zeros((1, 128), dtype=f32)

    gmax_ref[...] = jnp.maximum(gmax_ref[...], bmax)
    gmean_ref[...] = gmean_ref[...] + bsum

    @pl.when(i == G - 1)
    def _():
        gmean_ref[...] = gmean_ref[...] * (1.0 / k)


def _pool_call(xp, sc, ms, thr, k):
    BR = 2048
    G = NPAD // BR
    return pl.pallas_call(
        functools.partial(_pool_body, k=k, G=G),
        grid=(G,),
        in_specs=[
            pl.BlockSpec((BR, D), lambda i: (i, 0)),
            pl.BlockSpec((BR, 1), lambda i: (i, 0)),
            pl.BlockSpec((BR, 1), lambda i: (i, 0)),
            pl.BlockSpec((1, 128), lambda i: (0, 0)),
        ],
        out_specs=[
            pl.BlockSpec((BR, D), lambda i: (i, 0)),
            pl.BlockSpec((BR, 1), lambda i: (i, 0)),
            pl.BlockSpec((1, 128), lambda i: (0, 0)),
            pl.BlockSpec((1, 128), lambda i: (0, 0)),
        ],
        out_shape=[
            jax.ShapeDtypeStruct((NPAD, D), f32),
            jax.ShapeDtypeStruct((NPAD, 1), f32),
            jax.ShapeDtypeStruct((1, 128), f32),
            jax.ShapeDtypeStruct((1, 128), f32),
        ],
    )(xp, sc, ms, thr)


def _head_body(r_ref, w1_ref, b1_ref, w2_ref, b2_ref, w3_ref, b3_ref, o_ref):
    z = jnp.sum(r_ref[...], axis=0, keepdims=True)
    z = jnp.maximum(
        jnp.dot(z, w1_ref[...], preferred_element_type=f32) + b1_ref[...], 0.0)
    z = jnp.maximum(
        jnp.dot(z, w2_ref[...], preferred_element_type=f32) + b2_ref[...], 0.0)
    z = jnp.dot(z, w3_ref[...], preferred_element_type=f32) + b3_ref[...]
    m = jnp.max(z)
    zz = z - m
    o_ref[...] = zz - jnp.log(jnp.sum(jnp.exp(zz)))


def _head_call(R, lw1, lb1, lw2, lb2, lw3, lb3):
    return pl.pallas_call(
        _head_body,
        out_shape=jax.ShapeDtypeStruct((1, NUM_CLASSES), f32),
    )(R, lw1, lb1, lw2, lb2, lw3, lb3)


def kernel(x, edge_index, batch,
           W1, asrc1, adst1, b1, W2, asrc2, adst2, b2, W3, asrc3, adst3, b3,
           p1, p2, lw1, lb1, lw2, lb2, lw3, lb3):
    del batch

    X = jnp.concatenate([x.astype(f32), jnp.zeros((NPAD - N, D), f32)])
    act = jnp.concatenate([jnp.ones((N, 1), f32), jnp.zeros((NPAD - N, 1), f32)])

    src = edge_index[0].astype(jnp.int32)
    dst = edge_index[1].astype(jnp.int32)
    npd = EPAD - E
    pad_src = jnp.arange(npd, dtype=jnp.int32) % N
    pad_dst = (NPAD - 16) + (jnp.arange(npd, dtype=jnp.int32) % 16)
    src3 = jnp.concatenate([src, pad_src]).reshape(NW, NCHUNK, C)
    dst3 = jnp.concatenate([dst, pad_dst]).reshape(NW, NCHUNK, C)

    k1 = int(math.ceil(0.6 * N))
    k2 = int(math.ceil(0.6 * k1))
    k3 = int(math.ceil(0.6 * k2))
    layers = [
        (W1, asrc1, adst1, b1, p1, k1),
        (W2, asrc2, adst2, b2, p2, k2),
        (W3, asrc3, adst3, b3, p2, k3),
    ]

    readouts = []
    for (W, avs, avd, b, p, k) in layers:
        h, asm, adm, amax = _pre_call(X, W, avs.reshape(D, 1),
                                      avd.reshape(D, 1), act)
        acc2, den2 = _edge_call(h, asm.reshape(NPAD), adm.reshape(NPAD),
                                src3, dst3, amax.reshape(128)[:16])
        xp, sc, ms = _comb_call(acc2, den2.reshape(NC, NPAD, 1), h, asm, adm,
                                act, amax, b.reshape(1, D), p.reshape(D, 1))
        thr = _thr_call(ms.reshape(NPAD // 128, 128), k)
        X, act, gmax, gmean = _pool_call(xp, sc, ms, thr, k)
        readouts.append(jnp.concatenate([gmax, gmean], axis=1))

    R = jnp.concatenate([r for r in readouts], axis=0)
    out = _head_call(R, lw1, lb1.reshape(1, 128), lw2, lb2.reshape(1, 64),
                     lw3, lb3.reshape(1, NUM_CLASSES))
    return out


# speedup vs baseline: 36.1738x; 1.2083x over previous
import functools
import math

import jax
import jax.numpy as jnp
from jax import lax
from jax.experimental import pallas as pl
from jax.experimental.pallas import tpu as pltpu
from jax.experimental.pallas import tpu_sc as plsc

N = 10000
D = 128
E = 320000
NUM_CLASSES = 10

NPAD = 10240
NC, NS, L = 2, 16, 16
NW = NC * NS
C = 96
NCHUNK = -2 * ((E + NW * C - 1) // (NW * C) // -2)
EPAD = NCHUNK * NW * C
ETILE = NCHUNK * C
ROWS_T = NPAD // NS
NEG = -1e9

f32 = jnp.float32


def _pre_body(x_ref, w_ref, avs_ref, avd_ref, act_ref,
              h_ref, asm_ref, adm_ref, amax_ref):
    x = x_ref[...]
    h = jnp.dot(x, w_ref[...], preferred_element_type=f32)
    h_ref[...] = h
    a_s = jnp.dot(h, avs_ref[...], preferred_element_type=f32)
    a_d = jnp.dot(h, avd_ref[...], preferred_element_type=f32)
    act = act_ref[...]
    asm = jnp.where(act > 0, a_s, NEG)
    adm = jnp.where(act > 0, a_d, NEG)
    asm_ref[...] = asm
    adm_ref[...] = adm
    amax_ref[...] = jnp.full((1, 128), jnp.max(asm), dtype=f32)


def _pre_call(X, W, avs, avd, act):
    return pl.pallas_call(
        _pre_body,
        out_shape=[
            jax.ShapeDtypeStruct((NPAD, D), f32),
            jax.ShapeDtypeStruct((NPAD, 1), f32),
            jax.ShapeDtypeStruct((NPAD, 1), f32),
            jax.ShapeDtypeStruct((1, 128), f32),
        ],
    )(X, W, avs, avd, act)


def _edge_body(h_hbm, asm_hbm, adm_hbm, src_hbm, dst_hbm, amax_hbm,
               acc_out, den_out,
               as_v, ad_v, src_v, dst_v, rows_v, ex_v, zv, amax_v,
               acc_sh, den_sh, sem0, sem1):
    cid = lax.axis_index("c")
    sid = lax.axis_index("s")
    wid = sid * NC + cid
    sems = (sem0, sem1)

    pltpu.sync_copy(asm_hbm, as_v)
    pltpu.sync_copy(adm_hbm, ad_v)
    pltpu.sync_copy(amax_hbm, amax_v)

    z16 = jnp.zeros((16,), f32)

    def _zrow(r, carry):
        for v in range(8):
            rows_v[0, r, pl.ds(16 * v, 16)] = z16
        return carry
    lax.fori_loop(0, C, _zrow, 0)

    def _zz(i, carry):
        zv[pl.ds(i * 16, 16)] = z16
        return carry
    lax.fori_loop(0, ROWS_T // 16, _zz, 0)

    base = sid * ROWS_T
    nzf = ROWS_T // C
    nzr = ROWS_T - nzf * C

    def _zacc(j, carry):
        pltpu.sync_copy(rows_v.at[0], acc_sh.at[pl.ds(base + j * C, C)])
        return carry
    lax.fori_loop(0, nzf, _zacc, 0)
    if nzr:
        pltpu.sync_copy(rows_v.at[0].at[pl.ds(0, nzr)],
                        acc_sh.at[pl.ds(base + nzf * C, nzr)])
    pltpu.sync_copy(zv, den_sh.at[pl.ds(base, ROWS_T)])
    plsc.subcore_barrier()

    amax = amax_v[...]

    def _stage(j, b):
        pltpu.sync_copy(src_hbm.at[wid, j], src_v.at[b])
        pltpu.sync_copy(dst_hbm.at[wid, j], dst_v.at[b])
        pltpu.async_copy(h_hbm.at[src_v.at[b]], rows_v.at[b], sems[b])

    def _wait(b):
        pltpu.make_async_copy(h_hbm.at[src_v.at[b]], rows_v.at[b],
                              sems[b]).wait()

    def _process(b):
        for i in range(C // 16):
            sv = src_v[b, pl.ds(i * 16, 16)]
            dv = dst_v[b, pl.ds(i * 16, 16)]
            a_s = plsc.load_gather(as_v, [sv])
            a_d = plsc.load_gather(ad_v, [dv])
            z = a_s + a_d
            lg = jnp.where(z >= 0, z, 0.2 * z)
            zb = amax + a_d
            m = jnp.where(zb >= 0, zb, 0.2 * zb)
            ex = jnp.exp(lg - m)
            ex_v[pl.ds(i * 16, 16)] = ex

        pltpu.sync_copy(ex_v, den_sh.at[dst_v.at[b]], add=True)

        def _rower(r, rcarry):
            exr = plsc.load_gather(ex_v, [lax.broadcast(r, (16,))])
            for v in range(8):
                sl = pl.ds(16 * v, 16)
                rows_v[b, r, sl] = rows_v[b, r, sl] * exr
            return rcarry
        lax.fori_loop(0, C, _rower, 0)

        pltpu.sync_copy(rows_v.at[b], acc_sh.at[dst_v.at[b]], add=True)

    _stage(0, 0)
    npair = NCHUNK // 2

    def _pair(t, carry):
        j0 = 2 * t
        _wait(0)
        _stage(j0 + 1, 1)
        _process(0)
        _wait(1)

        @pl.when(t < npair - 1)
        def _():
            _stage(j0 + 2, 0)
        _process(1)
        return carry

    lax.fori_loop(0, npair, _pair, 0)
    plsc.subcore_barrier()

    pltpu.sync_copy(acc_sh.at[pl.ds(base, ROWS_T)],
                    acc_out.at[cid, pl.ds(base, ROWS_T)])
    pltpu.sync_copy(den_sh.at[pl.ds(base, ROWS_T)],
                    den_out.at[cid, pl.ds(base, ROWS_T)])


def _edge_call(h, asm_flat, adm_flat, src3, dst3, amax16):
    mesh = plsc.VectorSubcoreMesh(core_axis_name="c", subcore_axis_name="s",
                                  num_cores=NC, num_subcores=NS)
    fn = pl.kernel(
        _edge_body,
        out_type=[
            jax.ShapeDtypeStruct((NC, NPAD, D), f32),
            jax.ShapeDtypeStruct((NC, NPAD), f32),
        ],
        mesh=mesh,
        scratch_types=[
            pltpu.VMEM((NPAD,), f32),
            pltpu.VMEM((NPAD,), f32),
            pltpu.VMEM((2, C), jnp.int32),
            pltpu.VMEM((2, C), jnp.int32),
            pltpu.VMEM((2, C, D), f32),
            pltpu.VMEM((C,), f32),
            pltpu.VMEM((ROWS_T,), f32),
            pltpu.VMEM((16,), f32),
            pltpu.VMEM_SHARED((NPAD, D), f32),
            pltpu.VMEM_SHARED((NPAD,), f32),
            pltpu.SemaphoreType.DMA,
            pltpu.SemaphoreType.DMA,
        ],
        compiler_params=pltpu.CompilerParams(needs_layout_passes=False,
                                             use_tc_tiling_on_sc=False),
    )
    return fn(h, asm_flat, adm_flat, src3, dst3, amax16)


def _comb_body(acc_ref, den_ref, h_ref, asm_ref, adm_ref, act_ref,
               amax_ref, b_ref, p_ref, xp_ref, sc_ref, ms_ref):
    amax = amax_ref[0, 0]
    asm = asm_ref[...]
    adm = adm_ref[...]
    z = asm + adm
    lg = jnp.where(z >= 0, z, 0.2 * z)
    zb = amax + adm
    m = jnp.where(zb >= 0, zb, 0.2 * zb)
    exl = jnp.exp(lg - m)
    acc = acc_ref[0] + acc_ref[1]
    den = den_ref[0] + den_ref[1] + exl + 1e-16
    h = h_ref[...]
    out = (acc + exl * h) / den + b_ref[...]
    xp = jnp.maximum(out, 0.0)
    xp_ref[...] = xp
    p = p_ref[...]
    pn = jnp.sqrt(jnp.sum(p * p)) + 1e-16
    s = jnp.dot(xp, p, preferred_element_type=f32) / pn
    sc_ref[...] = s
    ms_ref[...] = jnp.where(act_ref[...] > 0, s, NEG)


def _comb_call(acc2, den2c, h, asm, adm, act, amax, b2d, pcol):
    BR = 2048
    G = NPAD // BR
    return pl.pallas_call(
        _comb_body,
        grid=(G,),
        in_specs=[
            pl.BlockSpec((NC, BR, D), lambda i: (0, i, 0)),
            pl.BlockSpec((NC, BR, 1), lambda i: (0, i, 0)),
            pl.BlockSpec((BR, D), lambda i: (i, 0)),
            pl.BlockSpec((BR, 1), lambda i: (i, 0)),
            pl.BlockSpec((BR, 1), lambda i: (i, 0)),
            pl.BlockSpec((BR, 1), lambda i: (i, 0)),
            pl.BlockSpec((1, 128), lambda i: (0, 0)),
            pl.BlockSpec((1, D), lambda i: (0, 0)),
            pl.BlockSpec((D, 1), lambda i: (0, 0)),
        ],
        out_specs=[
            pl.BlockSpec((BR, D), lambda i: (i, 0)),
            pl.BlockSpec((BR, 1), lambda i: (i, 0)),
            pl.BlockSpec((BR, 1), lambda i: (i, 0)),
        ],
        out_shape=[
            jax.ShapeDtypeStruct((NPAD, D), f32),
            jax.ShapeDtypeStruct((NPAD, 1), f32),
            jax.ShapeDtypeStruct((NPAD, 1), f32),
        ],
    )(acc2, den2c, h, asm, adm, act, amax, b2d, pcol)


def _thr_body(ms_ref, thr_ref, *, k):
    ms = ms_ref[...]
    lo0 = jnp.min(ms)
    hi0 = jnp.max(ms) + 1.0

    def _it(_, carry):
        lo, hi = carry
        mid = 0.5 * (lo + hi)
        cnt = jnp.sum(jnp.where(ms >= mid, 1.0, 0.0))
        ge = cnt >= k
        return (jnp.where(ge, mid, lo), jnp.where(ge, hi, mid))

    lo, hi = lax.fori_loop(0, 96, _it, (lo0, hi0))
    thr_ref[...] = jnp.full((1, 128), lo, dtype=f32)


def _thr_call(ms2d, k):
    return pl.pallas_call(
        functools.partial(_thr_body, k=k),
        out_shape=jax.ShapeDtypeStruct((1, 128), f32),
    )(ms2d)


def _pool_body(xp_ref, sc_ref, ms_ref, thr_ref,
               xn_ref, actn_ref, gmax_ref, gmean_ref, *, k, G):
    i = pl.program_id(0)
    thr = thr_ref[0, 0]
    kept = ms_ref[...] >= thr
    gate = jnp.where(kept, jnp.tanh(sc_ref[...]), 0.0)
    xn = xp_ref[...] * gate
    xn_ref[...] = xn
    actn_ref[...] = jnp.where(kept, 1.0, 0.0)
    xm = jnp.where(kept, xn, -1e30)
    bmax = jnp.max(xm, axis=0, keepdims=True)
    bsum = jnp.sum(xn, axis=0, keepdims=True)

    @pl.when(i == 0)
    def _():
        gmax_ref[...] = jnp.full((1, 128), -1e30, dtype=f32)
        gmean_ref[...] = jnp.zeros((1, 128), dtype=f32)

    gmax_ref[...] = jnp.maximum(gmax_ref[...], bmax)
    gmean_ref[...] = gmean_ref[...] + bsum

    @pl.when(i == G - 1)
    def _():
        gmean_ref[...] = gmean_ref[...] * (1.0 / k)


def _pool_call(xp, sc, ms, thr, k):
    BR = 2048
    G = NPAD // BR
    return pl.pallas_call(
        functools.partial(_pool_body, k=k, G=G),
        grid=(G,),
        in_specs=[
            pl.BlockSpec((BR, D), lambda i: (i, 0)),
            pl.BlockSpec((BR, 1), lambda i: (i, 0)),
            pl.BlockSpec((BR, 1), lambda i: (i, 0)),
            pl.BlockSpec((1, 128), lambda i: (0, 0)),
        ],
        out_specs=[
            pl.BlockSpec((BR, D), lambda i: (i, 0)),
            pl.BlockSpec((BR, 1), lambda i: (i, 0)),
            pl.BlockSpec((1, 128), lambda i: (0, 0)),
            pl.BlockSpec((1, 128), lambda i: (0, 0)),
        ],
        out_shape=[
            jax.ShapeDtypeStruct((NPAD, D), f32),
            jax.ShapeDtypeStruct((NPAD, 1), f32),
            jax.ShapeDtypeStruct((1, 128), f32),
            jax.ShapeDtypeStruct((1, 128), f32),
        ],
    )(xp, sc, ms, thr)


def _head_body(r_ref, w1_ref, b1_ref, w2_ref, b2_ref, w3_ref, b3_ref, o_ref):
    z = jnp.sum(r_ref[...], axis=0, keepdims=True)
    z = jnp.maximum(
        jnp.dot(z, w1_ref[...], preferred_element_type=f32) + b1_ref[...], 0.0)
    z = jnp.maximum(
        jnp.dot(z, w2_ref[...], preferred_element_type=f32) + b2_ref[...], 0.0)
    z = jnp.dot(z, w3_ref[...], preferred_element_type=f32) + b3_ref[...]
    m = jnp.max(z)
    zz = z - m
    o_ref[...] = zz - jnp.log(jnp.sum(jnp.exp(zz)))


def _head_call(R, lw1, lb1, lw2, lb2, lw3, lb3):
    return pl.pallas_call(
        _head_body,
        out_shape=jax.ShapeDtypeStruct((1, NUM_CLASSES), f32),
    )(R, lw1, lb1, lw2, lb2, lw3, lb3)


def kernel(x, edge_index, batch,
           W1, asrc1, adst1, b1, W2, asrc2, adst2, b2, W3, asrc3, adst3, b3,
           p1, p2, lw1, lb1, lw2, lb2, lw3, lb3):
    del batch

    X = jnp.concatenate([x.astype(f32), jnp.zeros((NPAD - N, D), f32)])
    act = jnp.concatenate([jnp.ones((N, 1), f32), jnp.zeros((NPAD - N, 1), f32)])

    src = edge_index[0].astype(jnp.int32)
    dst = edge_index[1].astype(jnp.int32)
    npd = EPAD - E
    pad_src = jnp.arange(npd, dtype=jnp.int32) % N
    pad_dst = (NPAD - 16) + (jnp.arange(npd, dtype=jnp.int32) % 16)
    src3 = jnp.concatenate([src, pad_src]).reshape(NW, NCHUNK, C)
    dst3 = jnp.concatenate([dst, pad_dst]).reshape(NW, NCHUNK, C)

    k1 = int(math.ceil(0.6 * N))
    k2 = int(math.ceil(0.6 * k1))
    k3 = int(math.ceil(0.6 * k2))
    layers = [
        (W1, asrc1, adst1, b1, p1, k1),
        (W2, asrc2, adst2, b2, p2, k2),
        (W3, asrc3, adst3, b3, p2, k3),
    ]

    readouts = []
    for (W, avs, avd, b, p, k) in layers:
        h, asm, adm, amax = _pre_call(X, W, avs.reshape(D, 1),
                                      avd.reshape(D, 1), act)
        acc2, den2 = _edge_call(h, asm.reshape(NPAD), adm.reshape(NPAD),
                                src3, dst3, amax.reshape(128)[:16])
        xp, sc, ms = _comb_call(acc2, den2.reshape(NC, NPAD, 1), h, asm, adm,
                                act, amax, b.reshape(1, D), p.reshape(D, 1))
        thr = _thr_call(ms.reshape(NPAD // 128, 128), k)
        X, act, gmax, gmean = _pool_call(xp, sc, ms, thr, k)
        readouts.append(jnp.concatenate([gmax, gmean], axis=1))

    R = jnp.concatenate([r for r in readouts], axis=0)
    out = _head_call(R, lw1, lb1.reshape(1, 128), lw2, lb2.reshape(1, 64),
                     lw3, lb3.reshape(1, NUM_CLASSES))
    return out


# speedup vs baseline: 37.0970x; 1.0255x over previous
import functools
import math

import jax
import jax.numpy as jnp
from jax import lax
from jax.experimental import pallas as pl
from jax.experimental.pallas import tpu as pltpu
from jax.experimental.pallas import tpu_sc as plsc

N = 10000
D = 128
E = 320000
NUM_CLASSES = 10

NPAD = 10240
NC, NS, L = 2, 16, 16
NW = NC * NS
C = 96
NCHUNK = -2 * ((E + NW * C - 1) // (NW * C) // -2)
EPAD = NCHUNK * NW * C
ETILE = NCHUNK * C
ROWS_T = NPAD // NS
NEG = -1e9

f32 = jnp.float32


def _pre_body(x_ref, w_ref, avs_ref, avd_ref, act_ref,
              h_ref, asm_ref, adm_ref, amax_ref):
    x = x_ref[...]
    h = jnp.dot(x, w_ref[...], preferred_element_type=f32)
    h_ref[...] = h
    a_s = jnp.dot(h, avs_ref[...], preferred_element_type=f32)
    a_d = jnp.dot(h, avd_ref[...], preferred_element_type=f32)
    act = act_ref[...]
    asm = jnp.where(act > 0, a_s, NEG)
    adm = jnp.where(act > 0, a_d, NEG)
    asm_ref[...] = asm
    adm_ref[...] = adm
    amax_ref[...] = jnp.full((1, 128), jnp.max(asm), dtype=f32)


def _pre_call(X, W, avs, avd, act):
    return pl.pallas_call(
        _pre_body,
        out_shape=[
            jax.ShapeDtypeStruct((NPAD, D), f32),
            jax.ShapeDtypeStruct((NPAD, 1), f32),
            jax.ShapeDtypeStruct((NPAD, 1), f32),
            jax.ShapeDtypeStruct((1, 128), f32),
        ],
    )(X, W, avs, avd, act)


def _edge_body(h_hbm, asm_hbm, adm_hbm, src_hbm, dst_hbm, amax_hbm,
               acc_out, den_out,
               as_v, ad_v, src_v, dst_v, rows_v, ex_v, zv, amax_v,
               acc_sh, den_sh, g0, g1, s0, s1, d0, d1):
    cid = lax.axis_index("c")
    sid = lax.axis_index("s")
    wid = sid * NC + cid
    gsems = (g0, g1)
    ssems = (s0, s1)
    dsems = (d0, d1)

    pltpu.sync_copy(asm_hbm, as_v)
    pltpu.sync_copy(adm_hbm, ad_v)
    pltpu.sync_copy(amax_hbm, amax_v)

    z16 = jnp.zeros((16,), f32)

    def _zrow(r, carry):
        for v in range(8):
            rows_v[0, r, pl.ds(16 * v, 16)] = z16
        return carry
    lax.fori_loop(0, C, _zrow, 0)

    def _zz(i, carry):
        zv[pl.ds(i * 16, 16)] = z16
        return carry
    lax.fori_loop(0, ROWS_T // 16, _zz, 0)

    base = sid * ROWS_T
    nzf = ROWS_T // C
    nzr = ROWS_T - nzf * C

    def _zacc(j, carry):
        pltpu.sync_copy(rows_v.at[0], acc_sh.at[pl.ds(base + j * C, C)])
        return carry
    lax.fori_loop(0, nzf, _zacc, 0)
    if nzr:
        pltpu.sync_copy(rows_v.at[0].at[pl.ds(0, nzr)],
                        acc_sh.at[pl.ds(base + nzf * C, nzr)])
    pltpu.sync_copy(zv, den_sh.at[pl.ds(base, ROWS_T)])
    plsc.subcore_barrier()

    amax = amax_v[...]

    def _stage_idx(j, b):
        pltpu.sync_copy(src_hbm.at[wid, j], src_v.at[b])
        pltpu.sync_copy(dst_hbm.at[wid, j], dst_v.at[b])

    def _gather(b):
        pltpu.async_copy(h_hbm.at[src_v.at[b]], rows_v.at[b], gsems[b])

    def _wait_gather(b):
        pltpu.make_async_copy(h_hbm.at[src_v.at[b]], rows_v.at[b],
                              gsems[b]).wait()

    def _wait_den(b):
        pltpu.make_async_copy(ex_v.at[b], den_sh.at[dst_v.at[b]],
                              dsems[b]).wait()

    def _wait_rows(b):
        pltpu.make_async_copy(rows_v.at[b], acc_sh.at[dst_v.at[b]],
                              ssems[b]).wait()

    def _process(b):
        for i in range(C // 16):
            sv = src_v[b, pl.ds(i * 16, 16)]
            dv = dst_v[b, pl.ds(i * 16, 16)]
            a_s = plsc.load_gather(as_v, [sv])
            a_d = plsc.load_gather(ad_v, [dv])
            z = a_s + a_d
            lg = jnp.where(z >= 0, z, 0.2 * z)
            zb = amax + a_d
            m = jnp.where(zb >= 0, zb, 0.2 * zb)
            ex = jnp.exp(lg - m)
            ex_v[b, pl.ds(i * 16, 16)] = ex

        pltpu.async_copy(ex_v.at[b], den_sh.at[dst_v.at[b]], dsems[b],
                         add=True)

        def _rower(r, rcarry):
            exr = plsc.load_gather(ex_v.at[b], [lax.broadcast(r, (16,))])
            for v in range(8):
                sl = pl.ds(16 * v, 16)
                rows_v[b, r, sl] = rows_v[b, r, sl] * exr
            return rcarry
        lax.fori_loop(0, C, _rower, 0)

        pltpu.async_copy(rows_v.at[b], acc_sh.at[dst_v.at[b]], ssems[b],
                         add=True)

    _stage_idx(0, 0)
    _gather(0)
    npair = NCHUNK // 2

    def _pair(t, carry):
        j0 = 2 * t
        _wait_gather(0)

        @pl.when(t > 0)
        def _():
            _wait_rows(1)
            _wait_den(1)
        _stage_idx(j0 + 1, 1)
        _gather(1)
        _process(0)

        _wait_gather(1)
        _wait_rows(0)
        _wait_den(0)

        @pl.when(t < npair - 1)
        def _():
            _stage_idx(j0 + 2, 0)
            _gather(0)
        _process(1)
        return carry

    lax.fori_loop(0, npair, _pair, 0)
    _wait_rows(1)
    _wait_den(1)
    plsc.subcore_barrier()

    pltpu.sync_copy(acc_sh.at[pl.ds(base, ROWS_T)],
                    acc_out.at[cid, pl.ds(base, ROWS_T)])
    pltpu.sync_copy(den_sh.at[pl.ds(base, ROWS_T)],
                    den_out.at[cid, pl.ds(base, ROWS_T)])


def _edge_call(h, asm_flat, adm_flat, src3, dst3, amax16):
    mesh = plsc.VectorSubcoreMesh(core_axis_name="c", subcore_axis_name="s",
                                  num_cores=NC, num_subcores=NS)
    fn = pl.kernel(
        _edge_body,
        out_type=[
            jax.ShapeDtypeStruct((NC, NPAD, D), f32),
            jax.ShapeDtypeStruct((NC, NPAD), f32),
        ],
        mesh=mesh,
        scratch_types=[
            pltpu.VMEM((NPAD,), f32),
            pltpu.VMEM((NPAD,), f32),
            pltpu.VMEM((2, C), jnp.int32),
            pltpu.VMEM((2, C), jnp.int32),
            pltpu.VMEM((2, C, D), f32),
            pltpu.VMEM((2, C), f32),
            pltpu.VMEM((ROWS_T,), f32),
            pltpu.VMEM((16,), f32),
            pltpu.VMEM_SHARED((NPAD, D), f32),
            pltpu.VMEM_SHARED((NPAD,), f32),
            pltpu.SemaphoreType.DMA,
            pltpu.SemaphoreType.DMA,
            pltpu.SemaphoreType.DMA,
            pltpu.SemaphoreType.DMA,
            pltpu.SemaphoreType.DMA,
            pltpu.SemaphoreType.DMA,
        ],
        compiler_params=pltpu.CompilerParams(needs_layout_passes=False,
                                             use_tc_tiling_on_sc=False),
    )
    return fn(h, asm_flat, adm_flat, src3, dst3, amax16)


def _comb_body(acc_ref, den_ref, h_ref, asm_ref, adm_ref, act_ref,
               amax_ref, b_ref, p_ref, xp_ref, sc_ref, ms_ref):
    amax = amax_ref[0, 0]
    asm = asm_ref[...]
    adm = adm_ref[...]
    z = asm + adm
    lg = jnp.where(z >= 0, z, 0.2 * z)
    zb = amax + adm
    m = jnp.where(zb >= 0, zb, 0.2 * zb)
    exl = jnp.exp(lg - m)
    acc = acc_ref[0] + acc_ref[1]
    den = den_ref[0] + den_ref[1] + exl + 1e-16
    h = h_ref[...]
    out = (acc + exl * h) / den + b_ref[...]
    xp = jnp.maximum(out, 0.0)
    xp_ref[...] = xp
    p = p_ref[...]
    pn = jnp.sqrt(jnp.sum(p * p)) + 1e-16
    s = jnp.dot(xp, p, preferred_element_type=f32) / pn
    sc_ref[...] = s
    ms_ref[...] = jnp.where(act_ref[...] > 0, s, NEG)


def _comb_call(acc2, den2c, h, asm, adm, act, amax, b2d, pcol):
    BR = 2048
    G = NPAD // BR
    return pl.pallas_call(
        _comb_body,
        grid=(G,),
        in_specs=[
            pl.BlockSpec((NC, BR, D), lambda i: (0, i, 0)),
            pl.BlockSpec((NC, BR, 1), lambda i: (0, i, 0)),
            pl.BlockSpec((BR, D), lambda i: (i, 0)),
            pl.BlockSpec((BR, 1), lambda i: (i, 0)),
            pl.BlockSpec((BR, 1), lambda i: (i, 0)),
            pl.BlockSpec((BR, 1), lambda i: (i, 0)),
            pl.BlockSpec((1, 128), lambda i: (0, 0)),
            pl.BlockSpec((1, D), lambda i: (0, 0)),
            pl.BlockSpec((D, 1), lambda i: (0, 0)),
        ],
        out_specs=[
            pl.BlockSpec((BR, D), lambda i: (i, 0)),
            pl.BlockSpec((BR, 1), lambda i: (i, 0)),
            pl.BlockSpec((BR, 1), lambda i: (i, 0)),
        ],
        out_shape=[
            jax.ShapeDtypeStruct((NPAD, D), f32),
            jax.ShapeDtypeStruct((NPAD, 1), f32),
            jax.ShapeDtypeStruct((NPAD, 1), f32),
        ],
    )(acc2, den2c, h, asm, adm, act, amax, b2d, pcol)


def _thr_body(ms_ref, thr_ref, *, k):
    ms = ms_ref[...]
    lo0 = jnp.min(ms)
    hi0 = jnp.max(ms) + 1.0

    def _it(_, carry):
        lo, hi = carry
        mid = 0.5 * (lo + hi)
        cnt = jnp.sum(jnp.where(ms >= mid, 1.0, 0.0))
        ge = cnt >= k
        return (jnp.where(ge, mid, lo), jnp.where(ge, hi, mid))

    lo, hi = lax.fori_loop(0, 96, _it, (lo0, hi0))
    thr_ref[...] = jnp.full((1, 128), lo, dtype=f32)


def _thr_call(ms2d, k):
    return pl.pallas_call(
        functools.partial(_thr_body, k=k),
        out_shape=jax.ShapeDtypeStruct((1, 128), f32),
    )(ms2d)


def _pool_body(xp_ref, sc_ref, ms_ref, thr_ref,
               xn_ref, actn_ref, gmax_ref, gmean_ref, *, k, G):
    i = pl.program_id(0)
    thr = thr_ref[0, 0]
    kept = ms_ref[...] >= thr
    gate = jnp.where(kept, jnp.tanh(sc_ref[...]), 0.0)
    xn = xp_ref[...] * gate
    xn_ref[...] = xn
    actn_ref[...] = jnp.where(kept, 1.0, 0.0)
    xm = jnp.where(kept, xn, -1e30)
    bmax = jnp.max(xm, axis=0, keepdims=True)
    bsum = jnp.sum(xn, axis=0, keepdims=True)

    @pl.when(i == 0)
    def _():
        gmax_ref[...] = jnp.full((1, 128), -1e30, dtype=f32)
        gmean_ref[...] = jnp.zeros((1, 128), dtype=f32)

    gmax_ref[...] = jnp.maximum(gmax_ref[...], bmax)
    gmean_ref[...] = gmean_ref[...] + bsum

    @pl.when(i == G - 1)
    def _():
        gmean_ref[...] = gmean_ref[...] * (1.0 / k)


def _pool_call(xp, sc, ms, thr, k):
    BR = 2048
    G = NPAD // BR
    return pl.pallas_call(
        functools.partial(_pool_body, k=k, G=G),
        grid=(G,),
        in_specs=[
            pl.BlockSpec((BR, D), lambda i: (i, 0)),
            pl.BlockSpec((BR, 1), lambda i: (i, 0)),
            pl.BlockSpec((BR, 1), lambda i: (i, 0)),
            pl.BlockSpec((1, 128), lambda i: (0, 0)),
        ],
        out_specs=[
            pl.BlockSpec((BR, D), lambda i: (i, 0)),
            pl.BlockSpec((BR, 1), lambda i: (i, 0)),
            pl.BlockSpec((1, 128), lambda i: (0, 0)),
            pl.BlockSpec((1, 128), lambda i: (0, 0)),
        ],
        out_shape=[
            jax.ShapeDtypeStruct((NPAD, D), f32),
            jax.ShapeDtypeStruct((NPAD, 1), f32),
            jax.ShapeDtypeStruct((1, 128), f32),
            jax.ShapeDtypeStruct((1, 128), f32),
        ],
    )(xp, sc, ms, thr)


def _head_body(r_ref, w1_ref, b1_ref, w2_ref, b2_ref, w3_ref, b3_ref, o_ref):
    z = jnp.sum(r_ref[...], axis=0, keepdims=True)
    z = jnp.maximum(
        jnp.dot(z, w1_ref[...], preferred_element_type=f32) + b1_ref[...], 0.0)
    z = jnp.maximum(
        jnp.dot(z, w2_ref[...], preferred_element_type=f32) + b2_ref[...], 0.0)
    z = jnp.dot(z, w3_ref[...], preferred_element_type=f32) + b3_ref[...]
    m = jnp.max(z)
    zz = z - m
    o_ref[...] = zz - jnp.log(jnp.sum(jnp.exp(zz)))


def _head_call(R, lw1, lb1, lw2, lb2, lw3, lb3):
    return pl.pallas_call(
        _head_body,
        out_shape=jax.ShapeDtypeStruct((1, NUM_CLASSES), f32),
    )(R, lw1, lb1, lw2, lb2, lw3, lb3)


def kernel(x, edge_index, batch,
           W1, asrc1, adst1, b1, W2, asrc2, adst2, b2, W3, asrc3, adst3, b3,
           p1, p2, lw1, lb1, lw2, lb2, lw3, lb3):
    del batch

    X = jnp.concatenate([x.astype(f32), jnp.zeros((NPAD - N, D), f32)])
    act = jnp.concatenate([jnp.ones((N, 1), f32), jnp.zeros((NPAD - N, 1), f32)])

    src = edge_index[0].astype(jnp.int32)
    dst = edge_index[1].astype(jnp.int32)
    npd = EPAD - E
    pad_src = jnp.arange(npd, dtype=jnp.int32) % N
    pad_dst = (NPAD - 16) + (jnp.arange(npd, dtype=jnp.int32) % 16)
    src3 = jnp.concatenate([src, pad_src]).reshape(NW, NCHUNK, C)
    dst3 = jnp.concatenate([dst, pad_dst]).reshape(NW, NCHUNK, C)

    k1 = int(math.ceil(0.6 * N))
    k2 = int(math.ceil(0.6 * k1))
    k3 = int(math.ceil(0.6 * k2))
    layers = [
        (W1, asrc1, adst1, b1, p1, k1),
        (W2, asrc2, adst2, b2, p2, k2),
        (W3, asrc3, adst3, b3, p2, k3),
    ]

    readouts = []
    for (W, avs, avd, b, p, k) in layers:
        h, asm, adm, amax = _pre_call(X, W, avs.reshape(D, 1),
                                      avd.reshape(D, 1), act)
        acc2, den2 = _edge_call(h, asm.reshape(NPAD), adm.reshape(NPAD),
                                src3, dst3, amax.reshape(128)[:16])
        xp, sc, ms = _comb_call(acc2, den2.reshape(NC, NPAD, 1), h, asm, adm,
                                act, amax, b.reshape(1, D), p.reshape(D, 1))
        thr = _thr_call(ms.reshape(NPAD // 128, 128), k)
        X, act, gmax, gmean = _pool_call(xp, sc, ms, thr, k)
        readouts.append(jnp.concatenate([gmax, gmean], axis=1))

    R = jnp.concatenate([r for r in readouts], axis=0)
    out = _head_call(R, lw1, lb1.reshape(1, 128), lw2, lb2.reshape(1, 64),
                     lw3, lb3.reshape(1, NUM_CLASSES))
    return out


# speedup vs baseline: 37.7855x; 1.0186x over previous
import functools
import math

import jax
import jax.numpy as jnp
from jax import lax
from jax.experimental import pallas as pl
from jax.experimental.pallas import tpu as pltpu
from jax.experimental.pallas import tpu_sc as plsc

N = 10000
D = 128
E = 320000
NUM_CLASSES = 10

NPAD = 10240
NC, NS, L = 2, 16, 16
NW = NC * NS
C = 96
NCHUNK = -2 * ((E + NW * C - 1) // (NW * C) // -2)
EPAD = NCHUNK * NW * C
ETILE = NCHUNK * C
ROWS_T = NPAD // NS
NEG = -1e9

f32 = jnp.float32


def _pre_body(x_ref, w_ref, avs_ref, avd_ref, act_ref,
              h_ref, asm_ref, adm_ref, amax_ref):
    x = x_ref[...]
    h = jnp.dot(x, w_ref[...], preferred_element_type=f32)
    h_ref[...] = h
    a_s = jnp.dot(h, avs_ref[...], preferred_element_type=f32)
    a_d = jnp.dot(h, avd_ref[...], preferred_element_type=f32)
    act = act_ref[...]
    asm = jnp.where(act > 0, a_s, NEG)
    adm = jnp.where(act > 0, a_d, NEG)
    asm_ref[...] = asm
    adm_ref[...] = adm
    amax_ref[...] = jnp.full((1, 128), jnp.max(asm), dtype=f32)


def _pre_call(X, W, avs, avd, act):
    return pl.pallas_call(
        _pre_body,
        out_shape=[
            jax.ShapeDtypeStruct((NPAD, D), f32),
            jax.ShapeDtypeStruct((NPAD, 1), f32),
            jax.ShapeDtypeStruct((NPAD, 1), f32),
            jax.ShapeDtypeStruct((1, 128), f32),
        ],
    )(X, W, avs, avd, act)


def _edge_body(h_hbm, asm_hbm, adm_hbm, src_hbm, dst_hbm, amax_hbm,
               acc_out, den_out,
               as_v, ad_v, src_v, dst_v, rows_v, ex_v, zv, amax_v,
               acc_sh, den_sh, g0, g1, s0, s1, d0, d1):
    cid = lax.axis_index("c")
    sid = lax.axis_index("s")
    wid = sid * NC + cid
    gsems = (g0, g1)
    ssems = (s0, s1)
    dsems = (d0, d1)

    pltpu.sync_copy(asm_hbm, as_v)
    pltpu.sync_copy(adm_hbm, ad_v)
    pltpu.sync_copy(amax_hbm, amax_v)

    z16 = jnp.zeros((16,), f32)

    def _zrow(r, carry):
        for v in range(8):
            rows_v[0, r, pl.ds(16 * v, 16)] = z16
        return carry
    lax.fori_loop(0, C, _zrow, 0)

    def _zz(i, carry):
        zv[pl.ds(i * 16, 16)] = z16
        return carry
    lax.fori_loop(0, ROWS_T // 16, _zz, 0)

    base = sid * ROWS_T
    nzf = ROWS_T // C
    nzr = ROWS_T - nzf * C

    def _zacc(j, carry):
        pltpu.sync_copy(rows_v.at[0], acc_sh.at[pl.ds(base + j * C, C)])
        return carry
    lax.fori_loop(0, nzf, _zacc, 0)
    if nzr:
        pltpu.sync_copy(rows_v.at[0].at[pl.ds(0, nzr)],
                        acc_sh.at[pl.ds(base + nzf * C, nzr)])
    pltpu.sync_copy(zv, den_sh.at[pl.ds(base, ROWS_T)])
    plsc.subcore_barrier()

    amax = amax_v[...]

    def _stage_idx(j, b):
        pltpu.sync_copy(src_hbm.at[wid, j], src_v.at[b])
        pltpu.sync_copy(dst_hbm.at[wid, j], dst_v.at[b])

    def _gather(b):
        pltpu.async_copy(h_hbm.at[src_v.at[b]], rows_v.at[b], gsems[b])

    def _wait_gather(b):
        pltpu.make_async_copy(h_hbm.at[src_v.at[b]], rows_v.at[b],
                              gsems[b]).wait()

    def _wait_den(b):
        pltpu.make_async_copy(ex_v.at[b], den_sh.at[dst_v.at[b]],
                              dsems[b]).wait()

    def _wait_rows(b):
        pltpu.make_async_copy(rows_v.at[b], acc_sh.at[dst_v.at[b]],
                              ssems[b]).wait()

    def _process(b):
        for i in range(C // 16):
            sv = src_v[b, pl.ds(i * 16, 16)]
            dv = dst_v[b, pl.ds(i * 16, 16)]
            a_s = plsc.load_gather(as_v, [sv])
            a_d = plsc.load_gather(ad_v, [dv])
            z = a_s + a_d
            lg = jnp.where(z >= 0, z, 0.2 * z)
            zb = amax + a_d
            m = jnp.where(zb >= 0, zb, 0.2 * zb)
            ex = jnp.exp(lg - m)
            ex_v[b, pl.ds(i * 16, 16)] = ex

        pltpu.async_copy(ex_v.at[b], den_sh.at[dst_v.at[b]], dsems[b],
                         add=True)

        def _rower(r, rcarry):
            exr = plsc.load_gather(ex_v.at[b], [lax.broadcast(r, (16,))])
            for v in range(8):
                sl = pl.ds(16 * v, 16)
                rows_v[b, r, sl] = rows_v[b, r, sl] * exr
            return rcarry
        lax.fori_loop(0, C, _rower, 0)

        pltpu.async_copy(rows_v.at[b], acc_sh.at[dst_v.at[b]], ssems[b],
                         add=True)

    _stage_idx(0, 0)
    _gather(0)
    npair = NCHUNK // 2

    def _pair(t, carry):
        j0 = 2 * t
        _wait_gather(0)

        @pl.when(t > 0)
        def _():
            _wait_rows(1)
            _wait_den(1)
        _stage_idx(j0 + 1, 1)
        _gather(1)
        _process(0)

        _wait_gather(1)
        _wait_rows(0)
        _wait_den(0)

        @pl.when(t < npair - 1)
        def _():
            _stage_idx(j0 + 2, 0)
            _gather(0)
        _process(1)
        return carry

    lax.fori_loop(0, npair, _pair, 0)
    _wait_rows(1)
    _wait_den(1)
    plsc.subcore_barrier()

    pltpu.sync_copy(acc_sh.at[pl.ds(base, ROWS_T)],
                    acc_out.at[cid, pl.ds(base, ROWS_T)])
    pltpu.sync_copy(den_sh.at[pl.ds(base, ROWS_T)],
                    den_out.at[cid, pl.ds(base, ROWS_T)])


def _edge_call(h, asm_flat, adm_flat, src3, dst3, amax16):
    mesh = plsc.VectorSubcoreMesh(core_axis_name="c", subcore_axis_name="s",
                                  num_cores=NC, num_subcores=NS)
    fn = pl.kernel(
        _edge_body,
        out_type=[
            jax.ShapeDtypeStruct((NC, NPAD, D), f32),
            jax.ShapeDtypeStruct((NC, NPAD), f32),
        ],
        mesh=mesh,
        scratch_types=[
            pltpu.VMEM((NPAD,), f32),
            pltpu.VMEM((NPAD,), f32),
            pltpu.VMEM((2, C), jnp.int32),
            pltpu.VMEM((2, C), jnp.int32),
            pltpu.VMEM((2, C, D), f32),
            pltpu.VMEM((2, C), f32),
            pltpu.VMEM((ROWS_T,), f32),
            pltpu.VMEM((16,), f32),
            pltpu.VMEM_SHARED((NPAD, D), f32),
            pltpu.VMEM_SHARED((NPAD,), f32),
            pltpu.SemaphoreType.DMA,
            pltpu.SemaphoreType.DMA,
            pltpu.SemaphoreType.DMA,
            pltpu.SemaphoreType.DMA,
            pltpu.SemaphoreType.DMA,
            pltpu.SemaphoreType.DMA,
        ],
        compiler_params=pltpu.CompilerParams(needs_layout_passes=False,
                                             use_tc_tiling_on_sc=False),
    )
    return fn(h, asm_flat, adm_flat, src3, dst3, amax16)


def _comb_body(acc_ref, den_ref, h_ref, asm_ref, adm_ref, act_ref,
               amax_ref, b_ref, p_ref, xp_ref, sc_ref, ms_ref):
    amax = amax_ref[0, 0]
    asm = asm_ref[...]
    adm = adm_ref[...]
    z = asm + adm
    lg = jnp.where(z >= 0, z, 0.2 * z)
    zb = amax + adm
    m = jnp.where(zb >= 0, zb, 0.2 * zb)
    exl = jnp.exp(lg - m)
    acc = acc_ref[0] + acc_ref[1]
    den = den_ref[0] + den_ref[1] + exl + 1e-16
    h = h_ref[...]
    out = (acc + exl * h) / den + b_ref[...]
    xp = jnp.maximum(out, 0.0)
    xp_ref[...] = xp
    p = p_ref[...]
    pn = jnp.sqrt(jnp.sum(p * p)) + 1e-16
    s = jnp.dot(xp, p, preferred_element_type=f32) / pn
    sc_ref[...] = s
    ms_ref[...] = jnp.where(act_ref[...] > 0, s, NEG)


def _comb_call(acc2, den2c, h, asm, adm, act, amax, b2d, pcol):
    BR = 2048
    G = NPAD // BR
    return pl.pallas_call(
        _comb_body,
        grid=(G,),
        in_specs=[
            pl.BlockSpec((NC, BR, D), lambda i: (0, i, 0)),
            pl.BlockSpec((NC, BR, 1), lambda i: (0, i, 0)),
            pl.BlockSpec((BR, D), lambda i: (i, 0)),
            pl.BlockSpec((BR, 1), lambda i: (i, 0)),
            pl.BlockSpec((BR, 1), lambda i: (i, 0)),
            pl.BlockSpec((BR, 1), lambda i: (i, 0)),
            pl.BlockSpec((1, 128), lambda i: (0, 0)),
            pl.BlockSpec((1, D), lambda i: (0, 0)),
            pl.BlockSpec((D, 1), lambda i: (0, 0)),
        ],
        out_specs=[
            pl.BlockSpec((BR, D), lambda i: (i, 0)),
            pl.BlockSpec((BR, 1), lambda i: (i, 0)),
            pl.BlockSpec((BR, 1), lambda i: (i, 0)),
        ],
        out_shape=[
            jax.ShapeDtypeStruct((NPAD, D), f32),
            jax.ShapeDtypeStruct((NPAD, 1), f32),
            jax.ShapeDtypeStruct((NPAD, 1), f32),
        ],
    )(acc2, den2c, h, asm, adm, act, amax, b2d, pcol)


def _bisect(ms2, k):
    lo0 = jnp.min(ms2)
    hi0 = jnp.max(ms2) + 1.0

    def _it(_, carry):
        lo, hi = carry
        mid = 0.5 * (lo + hi)
        cnt = jnp.sum(jnp.where(ms2 >= mid, 1.0, 0.0))
        ge = cnt >= k
        return (jnp.where(ge, mid, lo), jnp.where(ge, hi, mid))

    lo, hi = lax.fori_loop(0, 96, _it, (lo0, hi0))
    return lo


def _post_body(xp_ref, sc_ref, ms_ref, ms2_ref, wn_ref, avsn_ref, avdn_ref,
               hn_ref, asmn_ref, admn_ref, amaxn_ref, actn_ref,
               gmax_ref, gmean_ref, *, k):
    thr = _bisect(ms2_ref[...], k)
    kept = ms_ref[...] >= thr
    gate = jnp.where(kept, jnp.tanh(sc_ref[...]), 0.0)
    xn = xp_ref[...] * gate
    actn = jnp.where(kept, 1.0, 0.0)
    actn_ref[...] = actn
    xm = jnp.where(kept, xn, -1e30)
    gmax_ref[...] = jnp.max(xm, axis=0, keepdims=True)
    gmean_ref[...] = jnp.sum(xn, axis=0, keepdims=True) * (1.0 / k)
    hn = jnp.dot(xn, wn_ref[...], preferred_element_type=f32)
    hn_ref[...] = hn
    a_s = jnp.dot(hn, avsn_ref[...], preferred_element_type=f32)
    a_d = jnp.dot(hn, avdn_ref[...], preferred_element_type=f32)
    asmn = jnp.where(kept, a_s, NEG)
    asmn_ref[...] = asmn
    admn_ref[...] = jnp.where(kept, a_d, NEG)
    amaxn_ref[...] = jnp.full((1, 128), jnp.max(asmn), dtype=f32)


def _post_call(xp, sc, ms, ms2d, Wn, avsn, avdn, k):
    return pl.pallas_call(
        functools.partial(_post_body, k=k),
        out_shape=[
            jax.ShapeDtypeStruct((NPAD, D), f32),
            jax.ShapeDtypeStruct((NPAD, 1), f32),
            jax.ShapeDtypeStruct((NPAD, 1), f32),
            jax.ShapeDtypeStruct((1, 128), f32),
            jax.ShapeDtypeStruct((NPAD, 1), f32),
            jax.ShapeDtypeStruct((1, 128), f32),
            jax.ShapeDtypeStruct((1, 128), f32),
        ],
    )(xp, sc, ms, ms2d, Wn, avsn, avdn)


def _post3_body(xp_ref, sc_ref, ms_ref, ms2_ref, gx_ref, gm_ref,
                w1a_ref, w1b_ref, b1_ref, w2_ref, b2_ref, w3_ref, b3_ref,
                o_ref, *, k):
    thr = _bisect(ms2_ref[...], k)
    kept = ms_ref[...] >= thr
    gate = jnp.where(kept, jnp.tanh(sc_ref[...]), 0.0)
    xn = xp_ref[...] * gate
    xm = jnp.where(kept, xn, -1e30)
    zmax = gx_ref[...] + jnp.max(xm, axis=0, keepdims=True)
    zmean = gm_ref[...] + jnp.sum(xn, axis=0, keepdims=True) * (1.0 / k)
    z = (jnp.dot(zmax, w1a_ref[...], preferred_element_type=f32)
         + jnp.dot(zmean, w1b_ref[...], preferred_element_type=f32)
         + b1_ref[...])
    z = jnp.maximum(z, 0.0)
    z = jnp.maximum(
        jnp.dot(z, w2_ref[...], preferred_element_type=f32) + b2_ref[...], 0.0)
    z = jnp.dot(z, w3_ref[...], preferred_element_type=f32) + b3_ref[...]
    m = jnp.max(z)
    zz = z - m
    o_ref[...] = zz - jnp.log(jnp.sum(jnp.exp(zz)))


def _post3_call(xp, sc, ms, ms2d, gx, gm, w1a, w1b, b1, w2, b2, w3, b3, k):
    return pl.pallas_call(
        functools.partial(_post3_body, k=k),
        out_shape=jax.ShapeDtypeStruct((1, NUM_CLASSES), f32),
    )(xp, sc, ms, ms2d, gx, gm, w1a, w1b, b1, w2, b2, w3, b3)


def kernel(x, edge_index, batch,
           W1, asrc1, adst1, b1, W2, asrc2, adst2, b2, W3, asrc3, adst3, b3,
           p1, p2, lw1, lb1, lw2, lb2, lw3, lb3):
    del batch

    X = jnp.concatenate([x.astype(f32), jnp.zeros((NPAD - N, D), f32)])
    act = jnp.concatenate([jnp.ones((N, 1), f32), jnp.zeros((NPAD - N, 1), f32)])

    src = edge_index[0].astype(jnp.int32)
    dst = edge_index[1].astype(jnp.int32)
    npd = EPAD - E
    pad_src = jnp.arange(npd, dtype=jnp.int32) % N
    pad_dst = (NPAD - 16) + (jnp.arange(npd, dtype=jnp.int32) % 16)
    src3 = jnp.concatenate([src, pad_src]).reshape(NW, NCHUNK, C)
    dst3 = jnp.concatenate([dst, pad_dst]).reshape(NW, NCHUNK, C)

    k1 = int(math.ceil(0.6 * N))
    k2 = int(math.ceil(0.6 * k1))
    k3 = int(math.ceil(0.6 * k2))

    h, asm, adm, amax = _pre_call(X, W1, asrc1.reshape(D, 1),
                                  adst1.reshape(D, 1), act)
    gx = jnp.zeros((1, 128), f32)
    gm = jnp.zeros((1, 128), f32)

    layers = [
        (b1, p1, k1, W2, asrc2, adst2),
        (b2, p2, k2, W3, asrc3, adst3),
        (b3, p2, k3, None, None, None),
    ]
    for li, (b, p, k, Wn, avsn, avdn) in enumerate(layers):
        acc2, den2 = _edge_call(h, asm.reshape(NPAD), adm.reshape(NPAD),
                                src3, dst3, amax.reshape(128)[:16])
        xp, sc, ms = _comb_call(acc2, den2.reshape(NC, NPAD, 1), h, asm, adm,
                                act, amax, b.reshape(1, D), p.reshape(D, 1))
        ms2d = ms.reshape(NPAD // 128, 128)
        if li < 2:
            h, asm, adm, amax, act, gmax, gmean = _post_call(
                xp, sc, ms, ms2d, Wn, avsn.reshape(D, 1), avdn.reshape(D, 1),
                k)
            gx = gx + gmax
            gm = gm + gmean
        else:
            out = _post3_call(xp, sc, ms, ms2d, gx, gm,
                              lw1[:128], lw1[128:], lb1.reshape(1, 128),
                              lw2, lb2.reshape(1, 64),
                              lw3, lb3.reshape(1, NUM_CLASSES), k)
    return out


# speedup vs baseline: 50.3789x; 1.3333x over previous
import functools
import math

import jax
import jax.numpy as jnp
from jax import lax
from jax.experimental import pallas as pl
from jax.experimental.pallas import tpu as pltpu
from jax.experimental.pallas import tpu_sc as plsc

N = 10000
D = 128
E = 320000
NUM_CLASSES = 10

NPAD = 10240
NC, NS, L = 2, 16, 16
NW = NC * NS
C = 96
NCHUNK = -4 * ((E + NW * C - 1) // (NW * C) // -4)
EPAD = NCHUNK * NW * C
ETILE = NCHUNK * C
ROWS_T = NPAD // NS
NEG = -1e9

f32 = jnp.float32


def _pre_body(x_ref, w_ref, avs_ref, avd_ref, act_ref,
              h_ref, asm_ref, adm_ref, amax_ref):
    x = x_ref[...]
    h = jnp.dot(x, w_ref[...], preferred_element_type=f32)
    h_ref[...] = h
    a_s = jnp.dot(h, avs_ref[...], preferred_element_type=f32)
    a_d = jnp.dot(h, avd_ref[...], preferred_element_type=f32)
    act = act_ref[...]
    asm = jnp.where(act > 0, a_s, NEG)
    adm = jnp.where(act > 0, a_d, NEG)
    asm_ref[...] = asm
    adm_ref[...] = adm
    amax_ref[...] = jnp.full((1, 128), jnp.max(asm), dtype=f32)


def _pre_call(X, W, avs, avd, act):
    return pl.pallas_call(
        _pre_body,
        out_shape=[
            jax.ShapeDtypeStruct((NPAD, D), f32),
            jax.ShapeDtypeStruct((NPAD, 1), f32),
            jax.ShapeDtypeStruct((NPAD, 1), f32),
            jax.ShapeDtypeStruct((1, 128), f32),
        ],
    )(X, W, avs, avd, act)


def _edge_body(h_hbm, asm_hbm, adm_hbm, sd_hbm, amax_hbm,
               acc_out, den_out,
               as_v, ad_v, sd_v, rows_v, ex_v, zv, amax_v,
               acc_sh, den_sh,
               g0, g1, s0, s1, d0, d1, i0, i1, i2, i3):
    cid = lax.axis_index("c")
    sid = lax.axis_index("s")
    wid = sid * NC + cid
    gsems = (g0, g1)
    ssems = (s0, s1)
    dsems = (d0, d1)
    isems = (i0, i1, i2, i3)

    pltpu.sync_copy(asm_hbm, as_v)
    pltpu.sync_copy(adm_hbm, ad_v)
    pltpu.sync_copy(amax_hbm, amax_v)

    z16 = jnp.zeros((16,), f32)

    def _zrow(r, carry):
        for v in range(8):
            rows_v[0, r, pl.ds(16 * v, 16)] = z16
        return carry
    lax.fori_loop(0, C, _zrow, 0)

    def _zz(i, carry):
        zv[pl.ds(i * 16, 16)] = z16
        return carry
    lax.fori_loop(0, ROWS_T // 16, _zz, 0)

    base = sid * ROWS_T
    nzf = ROWS_T // C
    nzr = ROWS_T - nzf * C

    def _zacc(j, carry):
        pltpu.sync_copy(rows_v.at[0], acc_sh.at[pl.ds(base + j * C, C)])
        return carry
    lax.fori_loop(0, nzf, _zacc, 0)
    if nzr:
        pltpu.sync_copy(rows_v.at[0].at[pl.ds(0, nzr)],
                        acc_sh.at[pl.ds(base + nzf * C, nzr)])
    pltpu.sync_copy(zv, den_sh.at[pl.ds(base, ROWS_T)])
    plsc.subcore_barrier()

    amax = amax_v[...]

    def _stage(j, sl):
        pltpu.async_copy(sd_hbm.at[wid, j], sd_v.at[sl], isems[sl])

    def _wait_idx(j, sl):
        pltpu.make_async_copy(sd_hbm.at[wid, j], sd_v.at[sl],
                              isems[sl]).wait()

    def _gather(b, sl):
        pltpu.async_copy(h_hbm.at[sd_v.at[sl, 0]], rows_v.at[b], gsems[b])

    def _wait_gather(b, sl):
        pltpu.make_async_copy(h_hbm.at[sd_v.at[sl, 0]], rows_v.at[b],
                              gsems[b]).wait()

    def _wait_den(b, sl):
        pltpu.make_async_copy(ex_v.at[b], den_sh.at[sd_v.at[sl, 1]],
                              dsems[b]).wait()

    def _wait_rows(b, sl):
        pltpu.make_async_copy(rows_v.at[b], acc_sh.at[sd_v.at[sl, 1]],
                              ssems[b]).wait()

    def _process(b, sl):
        for i in range(C // 16):
            sv = sd_v[sl, 0, pl.ds(i * 16, 16)]
            dv = sd_v[sl, 1, pl.ds(i * 16, 16)]
            a_s = plsc.load_gather(as_v, [sv])
            a_d = plsc.load_gather(ad_v, [dv])
            z = a_s + a_d
            lg = jnp.where(z >= 0, z, 0.2 * z)
            zb = amax + a_d
            m = jnp.where(zb >= 0, zb, 0.2 * zb)
            ex = jnp.exp(lg - m)
            ex_v[b, pl.ds(i * 16, 16)] = ex

        pltpu.async_copy(ex_v.at[b], den_sh.at[sd_v.at[sl, 1]], dsems[b],
                         add=True)

        def _rower(r2, rcarry):
            for rr in range(2):
                r = 2 * r2 + rr
                exr = plsc.load_gather(ex_v.at[b], [lax.broadcast(r, (16,))])
                for v in range(8):
                    sl2 = pl.ds(16 * v, 16)
                    rows_v[b, r, sl2] = rows_v[b, r, sl2] * exr
            return rcarry
        lax.fori_loop(0, C // 2, _rower, 0)

        pltpu.async_copy(rows_v.at[b], acc_sh.at[sd_v.at[sl, 1]], ssems[b],
                         add=True)

    _stage(0, 0)
    _wait_idx(0, 0)
    _gather(0, 0)
    _stage(1, 1)
    nquad = NCHUNK // 4

    def _quad(q, carry):
        j0 = 4 * q
        for c in range(4):
            b = c % 2
            jj = j0 + c
            _wait_gather(b, c)
            if c == 0:
                @pl.when(q > 0)
                def _():
                    _wait_rows(1 - b, 3)
                    _wait_den(1 - b, 3)
            else:
                _wait_rows(1 - b, c - 1)
                _wait_den(1 - b, c - 1)
            cn = (c + 1) % 4
            if c < 3:
                _wait_idx(jj + 1, cn)
                _gather(1 - b, cn)
            else:
                @pl.when(q < nquad - 1)
                def _():
                    _wait_idx(jj + 1, cn)
                    _gather(1 - b, cn)
            c2 = (c + 2) % 4
            if c < 2:
                _stage(jj + 2, c2)
            else:
                @pl.when(q < nquad - 1)
                def _():
                    _stage(jj + 2, c2)
            _process(b, c)
        return carry

    lax.fori_loop(0, nquad, _quad, 0)
    _wait_rows(1, 3)
    _wait_den(1, 3)
    plsc.subcore_barrier()

    pltpu.sync_copy(acc_sh.at[pl.ds(base, ROWS_T)],
                    acc_out.at[cid, pl.ds(base, ROWS_T)])
    pltpu.sync_copy(den_sh.at[pl.ds(base, ROWS_T)],
                    den_out.at[cid, pl.ds(base, ROWS_T)])


def _edge_call(h, asm_flat, adm_flat, sd3, amax16):
    mesh = plsc.VectorSubcoreMesh(core_axis_name="c", subcore_axis_name="s",
                                  num_cores=NC, num_subcores=NS)
    fn = pl.kernel(
        _edge_body,
        out_type=[
            jax.ShapeDtypeStruct((NC, NPAD, D), f32),
            jax.ShapeDtypeStruct((NC, NPAD), f32),
        ],
        mesh=mesh,
        scratch_types=[
            pltpu.VMEM((NPAD,), f32),
            pltpu.VMEM((NPAD,), f32),
            pltpu.VMEM((4, 2, C), jnp.int32),
            pltpu.VMEM((2, C, D), f32),
            pltpu.VMEM((2, C), f32),
            pltpu.VMEM((ROWS_T,), f32),
            pltpu.VMEM((16,), f32),
            pltpu.VMEM_SHARED((NPAD, D), f32),
            pltpu.VMEM_SHARED((NPAD,), f32),
        ] + [pltpu.SemaphoreType.DMA] * 10,
        compiler_params=pltpu.CompilerParams(needs_layout_passes=False,
                                             use_tc_tiling_on_sc=False),
    )
    return fn(h, asm_flat, adm_flat, sd3, amax16)


def _comb_body(acc_ref, den_ref, h_ref, asm_ref, adm_ref, act_ref,
               amax_ref, b_ref, p_ref, xp_ref, sc_ref, ms_ref):
    amax = amax_ref[0, 0]
    asm = asm_ref[...]
    adm = adm_ref[...]
    z = asm + adm
    lg = jnp.where(z >= 0, z, 0.2 * z)
    zb = amax + adm
    m = jnp.where(zb >= 0, zb, 0.2 * zb)
    exl = jnp.exp(lg - m)
    acc = acc_ref[0] + acc_ref[1]
    den = den_ref[0] + den_ref[1] + exl + 1e-16
    h = h_ref[...]
    out = (acc + exl * h) / den + b_ref[...]
    xp = jnp.maximum(out, 0.0)
    xp_ref[...] = xp
    p = p_ref[...]
    pn = jnp.sqrt(jnp.sum(p * p)) + 1e-16
    s = jnp.dot(xp, p, preferred_element_type=f32) / pn
    sc_ref[...] = s
    ms_ref[...] = jnp.where(act_ref[...] > 0, s, NEG)


def _comb_call(acc2, den2c, h, asm, adm, act, amax, b2d, pcol):
    BR = 2048
    G = NPAD // BR
    return pl.pallas_call(
        _comb_body,
        grid=(G,),
        in_specs=[
            pl.BlockSpec((NC, BR, D), lambda i: (0, i, 0)),
            pl.BlockSpec((NC, BR, 1), lambda i: (0, i, 0)),
            pl.BlockSpec((BR, D), lambda i: (i, 0)),
            pl.BlockSpec((BR, 1), lambda i: (i, 0)),
            pl.BlockSpec((BR, 1), lambda i: (i, 0)),
            pl.BlockSpec((BR, 1), lambda i: (i, 0)),
            pl.BlockSpec((1, 128), lambda i: (0, 0)),
            pl.BlockSpec((1, D), lambda i: (0, 0)),
            pl.BlockSpec((D, 1), lambda i: (0, 0)),
        ],
        out_specs=[
            pl.BlockSpec((BR, D), lambda i: (i, 0)),
            pl.BlockSpec((BR, 1), lambda i: (i, 0)),
            pl.BlockSpec((BR, 1), lambda i: (i, 0)),
        ],
        out_shape=[
            jax.ShapeDtypeStruct((NPAD, D), f32),
            jax.ShapeDtypeStruct((NPAD, 1), f32),
            jax.ShapeDtypeStruct((NPAD, 1), f32),
        ],
    )(acc2, den2c, h, asm, adm, act, amax, b2d, pcol)


def _bisect(ms2, k):
    lo0 = jnp.min(ms2)
    hi0 = jnp.max(ms2) + 1.0

    def _it(_, carry):
        lo, hi = carry
        mid = 0.5 * (lo + hi)
        cnt = jnp.sum(jnp.where(ms2 >= mid, 1.0, 0.0))
        ge = cnt >= k
        return (jnp.where(ge, mid, lo), jnp.where(ge, hi, mid))

    lo, hi = lax.fori_loop(0, 96, _it, (lo0, hi0))
    return lo


def _post_body(xp_ref, sc_ref, ms_ref, ms2_ref, wn_ref, avsn_ref, avdn_ref,
               hn_ref, asmn_ref, admn_ref, amaxn_ref, actn_ref,
               gmax_ref, gmean_ref, *, k):
    thr = _bisect(ms2_ref[...], k)
    kept = ms_ref[...] >= thr
    gate = jnp.where(kept, jnp.tanh(sc_ref[...]), 0.0)
    xn = xp_ref[...] * gate
    actn = jnp.where(kept, 1.0, 0.0)
    actn_ref[...] = actn
    xm = jnp.where(kept, xn, -1e30)
    gmax_ref[...] = jnp.max(xm, axis=0, keepdims=True)
    gmean_ref[...] = jnp.sum(xn, axis=0, keepdims=True) * (1.0 / k)
    hn = jnp.dot(xn, wn_ref[...], preferred_element_type=f32)
    hn_ref[...] = hn
    a_s = jnp.dot(hn, avsn_ref[...], preferred_element_type=f32)
    a_d = jnp.dot(hn, avdn_ref[...], preferred_element_type=f32)
    asmn = jnp.where(kept, a_s, NEG)
    asmn_ref[...] = asmn
    admn_ref[...] = jnp.where(kept, a_d, NEG)
    amaxn_ref[...] = jnp.full((1, 128), jnp.max(asmn), dtype=f32)


def _post_call(xp, sc, ms, ms2d, Wn, avsn, avdn, k):
    return pl.pallas_call(
        functools.partial(_post_body, k=k),
        out_shape=[
            jax.ShapeDtypeStruct((NPAD, D), f32),
            jax.ShapeDtypeStruct((NPAD, 1), f32),
            jax.ShapeDtypeStruct((NPAD, 1), f32),
            jax.ShapeDtypeStruct((1, 128), f32),
            jax.ShapeDtypeStruct((NPAD, 1), f32),
            jax.ShapeDtypeStruct((1, 128), f32),
            jax.ShapeDtypeStruct((1, 128), f32),
        ],
    )(xp, sc, ms, ms2d, Wn, avsn, avdn)


def _post3_body(xp_ref, sc_ref, ms_ref, ms2_ref, gx_ref, gm_ref,
                w1a_ref, w1b_ref, b1_ref, w2_ref, b2_ref, w3_ref, b3_ref,
                o_ref, *, k):
    thr = _bisect(ms2_ref[...], k)
    kept = ms_ref[...] >= thr
    gate = jnp.where(kept, jnp.tanh(sc_ref[...]), 0.0)
    xn = xp_ref[...] * gate
    xm = jnp.where(kept, xn, -1e30)
    zmax = gx_ref[...] + jnp.max(xm, axis=0, keepdims=True)
    zmean = gm_ref[...] + jnp.sum(xn, axis=0, keepdims=True) * (1.0 / k)
    z = (jnp.dot(zmax, w1a_ref[...], preferred_element_type=f32)
         + jnp.dot(zmean, w1b_ref[...], preferred_element_type=f32)
         + b1_ref[...])
    z = jnp.maximum(z, 0.0)
    z = jnp.maximum(
        jnp.dot(z, w2_ref[...], preferred_element_type=f32) + b2_ref[...], 0.0)
    z = jnp.dot(z, w3_ref[...], preferred_element_type=f32) + b3_ref[...]
    m = jnp.max(z)
    zz = z - m
    o_ref[...] = zz - jnp.log(jnp.sum(jnp.exp(zz)))


def _post3_call(xp, sc, ms, ms2d, gx, gm, w1a, w1b, b1, w2, b2, w3, b3, k):
    return pl.pallas_call(
        functools.partial(_post3_body, k=k),
        out_shape=jax.ShapeDtypeStruct((1, NUM_CLASSES), f32),
    )(xp, sc, ms, ms2d, gx, gm, w1a, w1b, b1, w2, b2, w3, b3)


def kernel(x, edge_index, batch,
           W1, asrc1, adst1, b1, W2, asrc2, adst2, b2, W3, asrc3, adst3, b3,
           p1, p2, lw1, lb1, lw2, lb2, lw3, lb3):
    del batch

    X = jnp.concatenate([x.astype(f32), jnp.zeros((NPAD - N, D), f32)])
    act = jnp.concatenate([jnp.ones((N, 1), f32), jnp.zeros((NPAD - N, 1), f32)])

    src = edge_index[0].astype(jnp.int32)
    dst = edge_index[1].astype(jnp.int32)
    npd = EPAD - E
    pad_src = jnp.arange(npd, dtype=jnp.int32) % N
    pad_dst = (NPAD - 16) + (jnp.arange(npd, dtype=jnp.int32) % 16)
    src3 = jnp.concatenate([src, pad_src]).reshape(NW, NCHUNK, 1, C)
    dst3 = jnp.concatenate([dst, pad_dst]).reshape(NW, NCHUNK, 1, C)
    sd3 = jnp.concatenate([src3, dst3], axis=2)

    k1 = int(math.ceil(0.6 * N))
    k2 = int(math.ceil(0.6 * k1))
    k3 = int(math.ceil(0.6 * k2))

    h, asm, adm, amax = _pre_call(X, W1, asrc1.reshape(D, 1),
                                  adst1.reshape(D, 1), act)
    gx = jnp.zeros((1, 128), f32)
    gm = jnp.zeros((1, 128), f32)

    layers = [
        (b1, p1, k1, W2, asrc2, adst2),
        (b2, p2, k2, W3, asrc3, adst3),
        (b3, p2, k3, None, None, None),
    ]
    for li, (b, p, k, Wn, avsn, avdn) in enumerate(layers):
        acc2, den2 = _edge_call(h, asm.reshape(NPAD), adm.reshape(NPAD),
                                sd3, amax.reshape(128)[:16])
        xp, sc, ms = _comb_call(acc2, den2.reshape(NC, NPAD, 1), h, asm, adm,
                                act, amax, b.reshape(1, D), p.reshape(D, 1))
        ms2d = ms.reshape(NPAD // 128, 128)
        if li < 2:
            h, asm, adm, amax, act, gmax, gmean = _post_call(
                xp, sc, ms, ms2d, Wn, avsn.reshape(D, 1), avdn.reshape(D, 1),
                k)
            gx = gx + gmax
            gm = gm + gmean
        else:
            out = _post3_call(xp, sc, ms, ms2d, gx, gm,
                              lw1[:128], lw1[128:], lb1.reshape(1, 128),
                              lw2, lb2.reshape(1, 64),
                              lw3, lb3.reshape(1, NUM_CLASSES), k)
    return out


# speedup vs baseline: 50.6427x; 1.0052x over previous
import functools
import math

import jax
import jax.numpy as jnp
from jax import lax
from jax.experimental import pallas as pl
from jax.experimental.pallas import tpu as pltpu
from jax.experimental.pallas import tpu_sc as plsc

N = 10000
D = 128
E = 320000
NUM_CLASSES = 10

NPAD = 10240
NC, NS, L = 2, 16, 16
NW = NC * NS
C = 96
NCHUNK = -4 * ((E + NW * C - 1) // (NW * C) // -4)
EPAD = NCHUNK * NW * C
ETILE = NCHUNK * C
ROWS_T = NPAD // NS
NEG = -1e9

f32 = jnp.float32


def _pre_body(x_ref, w_ref, avs_ref, avd_ref, act_ref,
              h_ref, asm_ref, adm_ref, amax_ref):
    x = x_ref[...]
    h = jnp.dot(x, w_ref[...], preferred_element_type=f32)
    h_ref[...] = h
    a_s = jnp.dot(h, avs_ref[...], preferred_element_type=f32)
    a_d = jnp.dot(h, avd_ref[...], preferred_element_type=f32)
    act = act_ref[...]
    asm = jnp.where(act > 0, a_s, NEG)
    adm = jnp.where(act > 0, a_d, NEG)
    asm_ref[...] = asm
    adm_ref[...] = adm
    amax_ref[...] = jnp.full((1, 128), jnp.max(asm), dtype=f32)


def _pre_call(X, W, avs, avd, act):
    return pl.pallas_call(
        _pre_body,
        out_shape=[
            jax.ShapeDtypeStruct((NPAD, D), f32),
            jax.ShapeDtypeStruct((NPAD, 1), f32),
            jax.ShapeDtypeStruct((NPAD, 1), f32),
            jax.ShapeDtypeStruct((1, 128), f32),
        ],
    )(X, W, avs, avd, act)


def _edge_body(h_hbm, asm_hbm, adm_hbm, sd_hbm, amax_hbm,
               acc_out, den_out,
               as_v, ad_v, sd_v, rows_v, ex_v, zv, amax_v,
               acc_sh, den_sh,
               g0, g1, s0, s1, d0, d1, i0, i1, i2, i3):
    cid = lax.axis_index("c")
    sid = lax.axis_index("s")
    wid = sid * NC + cid
    gsems = (g0, g1)
    ssems = (s0, s1)
    dsems = (d0, d1)
    isems = (i0, i1, i2, i3)

    pltpu.sync_copy(asm_hbm, as_v)
    pltpu.sync_copy(adm_hbm, ad_v)
    pltpu.sync_copy(amax_hbm, amax_v)

    z16 = jnp.zeros((16,), f32)

    def _zrow(r, carry):
        for v in range(8):
            rows_v[0, r, pl.ds(16 * v, 16)] = z16
        return carry
    lax.fori_loop(0, C, _zrow, 0)

    def _zz(i, carry):
        zv[pl.ds(i * 16, 16)] = z16
        return carry
    lax.fori_loop(0, ROWS_T // 16, _zz, 0)

    base = sid * ROWS_T
    nzf = ROWS_T // C
    nzr = ROWS_T - nzf * C

    def _zacc(j, carry):
        pltpu.sync_copy(rows_v.at[0], acc_sh.at[pl.ds(base + j * C, C)])
        return carry
    lax.fori_loop(0, nzf, _zacc, 0)
    if nzr:
        pltpu.sync_copy(rows_v.at[0].at[pl.ds(0, nzr)],
                        acc_sh.at[pl.ds(base + nzf * C, nzr)])
    pltpu.sync_copy(zv, den_sh.at[pl.ds(base, ROWS_T)])
    plsc.subcore_barrier()

    amax = amax_v[...]

    def _stage(j, sl):
        pltpu.async_copy(sd_hbm.at[wid, j], sd_v.at[sl], isems[sl])

    def _wait_idx(j, sl):
        pltpu.make_async_copy(sd_hbm.at[wid, j], sd_v.at[sl],
                              isems[sl]).wait()

    def _gather(b, sl):
        pltpu.async_copy(h_hbm.at[sd_v.at[sl, 0]], rows_v.at[b], gsems[b])

    def _wait_gather(b, sl):
        pltpu.make_async_copy(h_hbm.at[sd_v.at[sl, 0]], rows_v.at[b],
                              gsems[b]).wait()

    def _wait_den(b, sl):
        pltpu.make_async_copy(ex_v.at[b], den_sh.at[sd_v.at[sl, 1]],
                              dsems[b]).wait()

    def _wait_rows(b, sl):
        pltpu.make_async_copy(rows_v.at[b], acc_sh.at[sd_v.at[sl, 1]],
                              ssems[b]).wait()

    def _process(b, sl):
        for i in range(C // 16):
            sv = sd_v[sl, 0, pl.ds(i * 16, 16)]
            dv = sd_v[sl, 1, pl.ds(i * 16, 16)]
            a_s = plsc.load_gather(as_v, [sv])
            a_d = plsc.load_gather(ad_v, [dv])
            z = a_s + a_d
            lg = jnp.where(z >= 0, z, 0.2 * z)
            zb = amax + a_d
            m = jnp.where(zb >= 0, zb, 0.2 * zb)
            ex = jnp.exp(lg - m)
            ex_v[b, pl.ds(i * 16, 16)] = ex

        pltpu.async_copy(ex_v.at[b], den_sh.at[sd_v.at[sl, 1]], dsems[b],
                         add=True)

        def _rower(r4, rcarry):
            for rr in range(4):
                r = 4 * r4 + rr
                exr = plsc.load_gather(ex_v.at[b], [lax.broadcast(r, (16,))])
                for v in range(8):
                    sl2 = pl.ds(16 * v, 16)
                    rows_v[b, r, sl2] = rows_v[b, r, sl2] * exr
            return rcarry
        lax.fori_loop(0, C // 4, _rower, 0)

        pltpu.async_copy(rows_v.at[b], acc_sh.at[sd_v.at[sl, 1]], ssems[b],
                         add=True)

    _stage(0, 0)
    _wait_idx(0, 0)
    _gather(0, 0)
    _stage(1, 1)
    nquad = NCHUNK // 4

    def _quad(q, carry):
        j0 = 4 * q
        for c in range(4):
            b = c % 2
            jj = j0 + c
            _wait_gather(b, c)
            if c == 0:
                @pl.when(q > 0)
                def _():
                    _wait_rows(1 - b, 3)
                    _wait_den(1 - b, 3)
            else:
                _wait_rows(1 - b, c - 1)
                _wait_den(1 - b, c - 1)
            cn = (c + 1) % 4
            if c < 3:
                _wait_idx(jj + 1, cn)
                _gather(1 - b, cn)
            else:
                @pl.when(q < nquad - 1)
                def _():
                    _wait_idx(jj + 1, cn)
                    _gather(1 - b, cn)
            c2 = (c + 2) % 4
            if c < 2:
                _stage(jj + 2, c2)
            else:
                @pl.when(q < nquad - 1)
                def _():
                    _stage(jj + 2, c2)
            _process(b, c)
        return carry

    lax.fori_loop(0, nquad, _quad, 0)
    _wait_rows(1, 3)
    _wait_den(1, 3)
    plsc.subcore_barrier()

    pltpu.sync_copy(acc_sh.at[pl.ds(base, ROWS_T)],
                    acc_out.at[cid, pl.ds(base, ROWS_T)])
    pltpu.sync_copy(den_sh.at[pl.ds(base, ROWS_T)],
                    den_out.at[cid, pl.ds(base, ROWS_T)])


def _edge_call(h, asm_flat, adm_flat, sd3, amax16):
    mesh = plsc.VectorSubcoreMesh(core_axis_name="c", subcore_axis_name="s",
                                  num_cores=NC, num_subcores=NS)
    fn = pl.kernel(
        _edge_body,
        out_type=[
            jax.ShapeDtypeStruct((NC, NPAD, D), f32),
            jax.ShapeDtypeStruct((NC, NPAD), f32),
        ],
        mesh=mesh,
        scratch_types=[
            pltpu.VMEM((NPAD,), f32),
            pltpu.VMEM((NPAD,), f32),
            pltpu.VMEM((4, 2, C), jnp.int32),
            pltpu.VMEM((2, C, D), f32),
            pltpu.VMEM((2, C), f32),
            pltpu.VMEM((ROWS_T,), f32),
            pltpu.VMEM((16,), f32),
            pltpu.VMEM_SHARED((NPAD, D), f32),
            pltpu.VMEM_SHARED((NPAD,), f32),
        ] + [pltpu.SemaphoreType.DMA] * 10,
        compiler_params=pltpu.CompilerParams(needs_layout_passes=False,
                                             use_tc_tiling_on_sc=False),
    )
    return fn(h, asm_flat, adm_flat, sd3, amax16)


def _comb_body(acc_ref, den_ref, h_ref, asm_ref, adm_ref, act_ref,
               amax_ref, b_ref, p_ref, xp_ref, sc_ref, ms_ref):
    amax = amax_ref[0, 0]
    asm = asm_ref[...]
    adm = adm_ref[...]
    z = asm + adm
    lg = jnp.where(z >= 0, z, 0.2 * z)
    zb = amax + adm
    m = jnp.where(zb >= 0, zb, 0.2 * zb)
    exl = jnp.exp(lg - m)
    acc = acc_ref[0] + acc_ref[1]
    den = den_ref[0] + den_ref[1] + exl + 1e-16
    h = h_ref[...]
    out = (acc + exl * h) / den + b_ref[...]
    xp = jnp.maximum(out, 0.0)
    xp_ref[...] = xp
    p = p_ref[...]
    pn = jnp.sqrt(jnp.sum(p * p)) + 1e-16
    s = jnp.dot(xp, p, preferred_element_type=f32) / pn
    sc_ref[...] = s
    ms_ref[...] = jnp.where(act_ref[...] > 0, s, NEG)


def _comb_call(acc2, den2c, h, asm, adm, act, amax, b2d, pcol):
    BR = 2048
    G = NPAD // BR
    return pl.pallas_call(
        _comb_body,
        grid=(G,),
        in_specs=[
            pl.BlockSpec((NC, BR, D), lambda i: (0, i, 0)),
            pl.BlockSpec((NC, BR, 1), lambda i: (0, i, 0)),
            pl.BlockSpec((BR, D), lambda i: (i, 0)),
            pl.BlockSpec((BR, 1), lambda i: (i, 0)),
            pl.BlockSpec((BR, 1), lambda i: (i, 0)),
            pl.BlockSpec((BR, 1), lambda i: (i, 0)),
            pl.BlockSpec((1, 128), lambda i: (0, 0)),
            pl.BlockSpec((1, D), lambda i: (0, 0)),
            pl.BlockSpec((D, 1), lambda i: (0, 0)),
        ],
        out_specs=[
            pl.BlockSpec((BR, D), lambda i: (i, 0)),
            pl.BlockSpec((BR, 1), lambda i: (i, 0)),
            pl.BlockSpec((BR, 1), lambda i: (i, 0)),
        ],
        out_shape=[
            jax.ShapeDtypeStruct((NPAD, D), f32),
            jax.ShapeDtypeStruct((NPAD, 1), f32),
            jax.ShapeDtypeStruct((NPAD, 1), f32),
        ],
    )(acc2, den2c, h, asm, adm, act, amax, b2d, pcol)


def _bisect(ms2, k):
    lo0 = jnp.min(ms2)
    hi0 = jnp.max(ms2) + 1.0

    def _it(_, carry):
        lo, hi = carry
        mid = 0.5 * (lo + hi)
        cnt = jnp.sum(jnp.where(ms2 >= mid, 1.0, 0.0))
        ge = cnt >= k
        return (jnp.where(ge, mid, lo), jnp.where(ge, hi, mid))

    lo, hi = lax.fori_loop(0, 96, _it, (lo0, hi0))
    return lo


def _post_body(xp_ref, sc_ref, ms_ref, ms2_ref, wn_ref, avsn_ref, avdn_ref,
               hn_ref, asmn_ref, admn_ref, amaxn_ref, actn_ref,
               gmax_ref, gmean_ref, *, k):
    thr = _bisect(ms2_ref[...], k)
    kept = ms_ref[...] >= thr
    gate = jnp.where(kept, jnp.tanh(sc_ref[...]), 0.0)
    xn = xp_ref[...] * gate
    actn = jnp.where(kept, 1.0, 0.0)
    actn_ref[...] = actn
    xm = jnp.where(kept, xn, -1e30)
    gmax_ref[...] = jnp.max(xm, axis=0, keepdims=True)
    gmean_ref[...] = jnp.sum(xn, axis=0, keepdims=True) * (1.0 / k)
    hn = jnp.dot(xn, wn_ref[...], preferred_element_type=f32)
    hn_ref[...] = hn
    a_s = jnp.dot(hn, avsn_ref[...], preferred_element_type=f32)
    a_d = jnp.dot(hn, avdn_ref[...], preferred_element_type=f32)
    asmn = jnp.where(kept, a_s, NEG)
    asmn_ref[...] = asmn
    admn_ref[...] = jnp.where(kept, a_d, NEG)
    amaxn_ref[...] = jnp.full((1, 128), jnp.max(asmn), dtype=f32)


def _post_call(xp, sc, ms, ms2d, Wn, avsn, avdn, k):
    return pl.pallas_call(
        functools.partial(_post_body, k=k),
        out_shape=[
            jax.ShapeDtypeStruct((NPAD, D), f32),
            jax.ShapeDtypeStruct((NPAD, 1), f32),
            jax.ShapeDtypeStruct((NPAD, 1), f32),
            jax.ShapeDtypeStruct((1, 128), f32),
            jax.ShapeDtypeStruct((NPAD, 1), f32),
            jax.ShapeDtypeStruct((1, 128), f32),
            jax.ShapeDtypeStruct((1, 128), f32),
        ],
    )(xp, sc, ms, ms2d, Wn, avsn, avdn)


def _post3_body(xp_ref, sc_ref, ms_ref, ms2_ref, gx_ref, gm_ref,
                w1a_ref, w1b_ref, b1_ref, w2_ref, b2_ref, w3_ref, b3_ref,
                o_ref, *, k):
    thr = _bisect(ms2_ref[...], k)
    kept = ms_ref[...] >= thr
    gate = jnp.where(kept, jnp.tanh(sc_ref[...]), 0.0)
    xn = xp_ref[...] * gate
    xm = jnp.where(kept, xn, -1e30)
    zmax = gx_ref[...] + jnp.max(xm, axis=0, keepdims=True)
    zmean = gm_ref[...] + jnp.sum(xn, axis=0, keepdims=True) * (1.0 / k)
    z = (jnp.dot(zmax, w1a_ref[...], preferred_element_type=f32)
         + jnp.dot(zmean, w1b_ref[...], preferred_element_type=f32)
         + b1_ref[...])
    z = jnp.maximum(z, 0.0)
    z = jnp.maximum(
        jnp.dot(z, w2_ref[...], preferred_element_type=f32) + b2_ref[...], 0.0)
    z = jnp.dot(z, w3_ref[...], preferred_element_type=f32) + b3_ref[...]
    m = jnp.max(z)
    zz = z - m
    o_ref[...] = zz - jnp.log(jnp.sum(jnp.exp(zz)))


def _post3_call(xp, sc, ms, ms2d, gx, gm, w1a, w1b, b1, w2, b2, w3, b3, k):
    return pl.pallas_call(
        functools.partial(_post3_body, k=k),
        out_shape=jax.ShapeDtypeStruct((1, NUM_CLASSES), f32),
    )(xp, sc, ms, ms2d, gx, gm, w1a, w1b, b1, w2, b2, w3, b3)


def kernel(x, edge_index, batch,
           W1, asrc1, adst1, b1, W2, asrc2, adst2, b2, W3, asrc3, adst3, b3,
           p1, p2, lw1, lb1, lw2, lb2, lw3, lb3):
    del batch

    X = jnp.concatenate([x.astype(f32), jnp.zeros((NPAD - N, D), f32)])
    act = jnp.concatenate([jnp.ones((N, 1), f32), jnp.zeros((NPAD - N, 1), f32)])

    src = edge_index[0].astype(jnp.int32)
    dst = edge_index[1].astype(jnp.int32)
    npd = EPAD - E
    pad_src = jnp.arange(npd, dtype=jnp.int32) % N
    pad_dst = (NPAD - 16) + (jnp.arange(npd, dtype=jnp.int32) % 16)
    src3 = jnp.concatenate([src, pad_src]).reshape(NW, NCHUNK, 1, C)
    dst3 = jnp.concatenate([dst, pad_dst]).reshape(NW, NCHUNK, 1, C)
    sd3 = jnp.concatenate([src3, dst3], axis=2)

    k1 = int(math.ceil(0.6 * N))
    k2 = int(math.ceil(0.6 * k1))
    k3 = int(math.ceil(0.6 * k2))

    h, asm, adm, amax = _pre_call(X, W1, asrc1.reshape(D, 1),
                                  adst1.reshape(D, 1), act)
    gx = jnp.zeros((1, 128), f32)
    gm = jnp.zeros((1, 128), f32)

    layers = [
        (b1, p1, k1, W2, asrc2, adst2),
        (b2, p2, k2, W3, asrc3, adst3),
        (b3, p2, k3, None, None, None),
    ]
    for li, (b, p, k, Wn, avsn, avdn) in enumerate(layers):
        acc2, den2 = _edge_call(h, asm.reshape(NPAD), adm.reshape(NPAD),
                                sd3, amax.reshape(128)[:16])
        xp, sc, ms = _comb_call(acc2, den2.reshape(NC, NPAD, 1), h, asm, adm,
                                act, amax, b.reshape(1, D), p.reshape(D, 1))
        ms2d = ms.reshape(NPAD // 128, 128)
        if li < 2:
            h, asm, adm, amax, act, gmax, gmean = _post_call(
                xp, sc, ms, ms2d, Wn, avsn.reshape(D, 1), avdn.reshape(D, 1),
                k)
            gx = gx + gmax
            gm = gm + gmean
        else:
            out = _post3_call(xp, sc, ms, ms2d, gx, gm,
                              lw1[:128], lw1[128:], lb1.reshape(1, 128),
                              lw2, lb2.reshape(1, 64),
                              lw3, lb3.reshape(1, NUM_CLASSES), k)
    return out
